# Initial kernel scaffold; baseline (speedup 1.0000x reference)
import jax
import jax.numpy as jnp
from jax.experimental import pallas as pl


def kernel(x, h, c, batch, edge_index, t, hW1, hb1, hW2, hb2, dW1, db1, dW2, db2, tW1, tb1, tW2, tb2, cW1, cb1, cW2, cb2, sW1, sb1, sW2, sb2, oW1, ob1, mW1, mb1, mW2, mb2, mW3, mb3, mW4, mb4):
    raise NotImplementedError("write your pallas kernel here")


# speedup vs baseline: 4.1233x; 4.1233x over previous
import functools

import jax
import jax.numpy as jnp
from jax import lax
from jax.experimental import pallas as pl
from jax.experimental.pallas import tpu as pltpu
from jax.experimental.pallas import tpu_sc as plsc

N = 10000
E = 320000
G = 64
H = 128
M = 32
NORM = 100.0

NC = 2
NS = 16
NW = NC * NS
CH = 128
NCHUNK = E // CH
NTRIP = (NCHUNK + NW - 1) // NW
ROWS_PER_TILE = N // NS
PW = 8

_INTERPRET = False


def _silu(v):
    return v * jax.nn.sigmoid(v)


def _prep_body(h_ref, bf_ref, t_ref, c_ref,
               hW1, hb1, hW2, hb2, tW1, tb1, tW2, tb2, cW1, cb1, cW2, cb2,
               oW1, ob1, A0, B0,
               hb_ref, nA_ref, nB_ref, tcco_ref):
    f32 = jnp.float32
    tcc = (_silu(jnp.dot(t_ref[...], tW1[...], preferred_element_type=f32) + tb1[...])
           @ tW2[...] + tb2[...])
    tcc = tcc + (_silu(jnp.dot(c_ref[...], cW1[...], preferred_element_type=f32) + cb1[...])
                 @ cW2[...] + cb2[...])
    h0 = (_silu(jnp.dot(h_ref[...], hW1[...], preferred_element_type=f32) + hb1[...])
          @ hW2[...] + hb2[...])
    gids = lax.broadcasted_iota(jnp.int32, (h_ref.shape[0], G), 1).astype(f32)
    oh = (bf_ref[...] == gids).astype(f32)
    hb = h0 + jnp.dot(oh, tcc, preferred_element_type=f32)
    hb_ref[...] = hb
    tcco_ref[...] = jnp.dot(oh, jnp.dot(tcc, oW1[...], preferred_element_type=f32),
                            preferred_element_type=f32)
    nA_ref[...] = jnp.dot(hb, A0[...], preferred_element_type=f32)
    nB_ref[...] = jnp.dot(hb, B0[...], preferred_element_type=f32)


def _run_prep(h, batchf, t, c, ws):
    f32 = jnp.float32
    nb = 10
    blk = N // nb
    full = lambda s: pl.BlockSpec(s, lambda i: (0,) * len(s))
    return pl.pallas_call(
        _prep_body,
        grid=(nb,),
        in_specs=[
            pl.BlockSpec((blk, 5), lambda i: (i, 0)),
            pl.BlockSpec((blk, 1), lambda i: (i, 0)),
            full((G, 6)), full((G, 5)),
            full((5, H)), full((1, H)), full((H, H)), full((1, H)),
            full((6, H)), full((1, H)), full((H, H)), full((1, H)),
            full((5, H)), full((1, H)), full((H, H)), full((1, H)),
            full((H, 5)), full((1, 5)), full((H, M)), full((H, M)),
        ],
        out_specs=[
            pl.BlockSpec((blk, H), lambda i: (i, 0)),
            pl.BlockSpec((blk, M), lambda i: (i, 0)),
            pl.BlockSpec((blk, M), lambda i: (i, 0)),
            pl.BlockSpec((blk, 5), lambda i: (i, 0)),
        ],
        out_shape=[
            jax.ShapeDtypeStruct((N, H), f32),
            jax.ShapeDtypeStruct((N, M), f32),
            jax.ShapeDtypeStruct((N, M), f32),
            jax.ShapeDtypeStruct((N, 5), f32),
        ],
        interpret=_INTERPRET,
    )(h, batchf, t, c, *ws)


def _gather_body(nA, nB, x0, x1, x2, rowi, coli,
                 gA, gB, dif,
                 x0v, x1v, x2v, ibr, ibc, bufA, bufB, bufD, sem1, sem2):
    wid = lax.axis_index("s") * NC + lax.axis_index("c")
    pltpu.sync_copy(x0, x0v)
    pltpu.sync_copy(x1, x1v)
    pltpu.sync_copy(x2, x2v)
    iota = lax.iota(jnp.int32, 16)

    def body(j, carry):
        cidx = wid + NW * j

        @pl.when(cidx < NCHUNK)
        def _():
            base = cidx * CH
            pltpu.sync_copy(rowi.at[pl.ds(base, CH)], ibr)
            pltpu.sync_copy(coli.at[pl.ds(base, CH)], ibc)
            cpA = pltpu.async_copy(nA.at[ibr], bufA, sem1)
            cpB = pltpu.async_copy(nB.at[ibc], bufB, sem2)
            for i in range(CH // 16):
                r16 = ibr[pl.ds(i * 16, 16)]
                c16 = ibc[pl.ds(i * 16, 16)]
                d0 = plsc.load_gather(x0v, [r16]) - plsc.load_gather(x0v, [c16])
                d1 = plsc.load_gather(x1v, [r16]) - plsc.load_gather(x1v, [c16])
                d2 = plsc.load_gather(x2v, [r16]) - plsc.load_gather(x2v, [c16])
                sq = d0 * d0 + d1 * d1 + d2 * d2
                er = iota + (i * 16)
                for comp, val in ((0, d0), (1, d1), (2, d2), (3, sq)):
                    cc = jnp.full((16,), comp, jnp.int32)
                    plsc.store_scatter(bufD, [er, cc], val)
            cpA.wait()
            cpB.wait()
            pltpu.sync_copy(bufA, gA.at[pl.ds(base, CH)])
            pltpu.sync_copy(bufB, gB.at[pl.ds(base, CH)])
            pltpu.sync_copy(bufD, dif.at[pl.ds(base, CH)])

        return carry

    lax.fori_loop(0, NTRIP, body, 0)


def _run_gather(nA, nB, x0, x1, x2, rowi, coli):
    f32 = jnp.float32
    fn = pl.kernel(
        _gather_body,
        out_type=[
            jax.ShapeDtypeStruct((E, M), f32),
            jax.ShapeDtypeStruct((E, M), f32),
            jax.ShapeDtypeStruct((E, 4), f32),
        ],
        mesh=plsc.VectorSubcoreMesh(core_axis_name="c", subcore_axis_name="s", num_cores=NC, num_subcores=NS),
        compiler_params=pltpu.CompilerParams(needs_layout_passes=False, use_tc_tiling_on_sc=False),
        scratch_types=[
            pltpu.VMEM((N,), f32), pltpu.VMEM((N,), f32), pltpu.VMEM((N,), f32),
            pltpu.VMEM((CH,), jnp.int32), pltpu.VMEM((CH,), jnp.int32),
            pltpu.VMEM((CH, M), f32), pltpu.VMEM((CH, M), f32),
            pltpu.VMEM((CH, 4), f32),
            pltpu.SemaphoreType.DMA, pltpu.SemaphoreType.DMA,
        ],
    )
    return fn(nA, nB, x0, x1, x2, rowi, coli)


def _edge_body(gA, gB, dif, dW1r, db1r, Dfold, bfold, mW2r, mb2r, mW3r, mb3r,
               S1f, sb1f, sW2r, sb2r, z_ref, pw_ref):
    f32 = jnp.float32
    df = dif[...]
    sqd = df[:, 3:4]
    dist = jnp.sqrt(jnp.maximum(sqd, 1e-12))
    d2 = _silu(dist * dW1r[...] + db1r[...])
    pre = (gA[...] + gB[...] + jnp.dot(d2, Dfold[...], preferred_element_type=f32)
           + bfold[...])
    m1 = _silu(pre)
    m2 = _silu(jnp.dot(m1, mW2r[...], preferred_element_type=f32) + mb2r[...])
    z = _silu(jnp.dot(m2, mW3r[...], preferred_element_type=f32) + mb3r[...])
    u = _silu(jnp.dot(z, S1f[...], preferred_element_type=f32) + sb1f[...])
    w = jnp.tanh(jnp.sum(u * sW2r[...], axis=1, keepdims=True) + sb2r[...])
    z_ref[...] = z
    pw_ref[...] = jnp.concatenate(
        [df[:, :3] * w, jnp.ones((df.shape[0], 1), f32),
         jnp.zeros((df.shape[0], PW - 4), f32)], axis=1)


def _run_edge(gA, gB, dif, ws):
    f32 = jnp.float32
    nb = 80
    blk = E // nb
    full = lambda s: pl.BlockSpec(s, lambda i: (0,) * len(s))
    return pl.pallas_call(
        _edge_body,
        grid=(nb,),
        in_specs=[
            pl.BlockSpec((blk, M), lambda i: (i, 0)),
            pl.BlockSpec((blk, M), lambda i: (i, 0)),
            pl.BlockSpec((blk, 4), lambda i: (i, 0)),
            full((1, M)), full((1, M)), full((M, M)), full((1, M)),
            full((M, M)), full((1, M)), full((M, M)), full((1, M)),
            full((M, H)), full((1, H)), full((1, H)), full((1, 1)),
        ],
        out_specs=[
            pl.BlockSpec((blk, M), lambda i: (i, 0)),
            pl.BlockSpec((blk, PW), lambda i: (i, 0)),
        ],
        out_shape=[
            jax.ShapeDtypeStruct((E, M), f32),
            jax.ShapeDtypeStruct((E, PW), f32),
        ],
        interpret=_INTERPRET,
    )(gA, gB, dif, *ws)


def _scatter_body(rowi, z, pw, zz, zp,
                  pz, pp,
                  accZ, accP, ib, zbuf, pbuf):
    cid = lax.axis_index("c")
    sid = lax.axis_index("s")
    wid = sid * NC + cid
    rbase = sid * ROWS_PER_TILE
    pltpu.sync_copy(zz.at[pl.ds(rbase, ROWS_PER_TILE)],
                    accZ.at[pl.ds(rbase, ROWS_PER_TILE)])
    pltpu.sync_copy(zp.at[pl.ds(rbase, ROWS_PER_TILE)],
                    accP.at[pl.ds(rbase, ROWS_PER_TILE)])
    plsc.subcore_barrier()

    def body(j, carry):
        cidx = wid + NW * j

        @pl.when(cidx < NCHUNK)
        def _():
            base = cidx * CH
            pltpu.sync_copy(rowi.at[pl.ds(base, CH)], ib)
            pltpu.sync_copy(z.at[pl.ds(base, CH)], zbuf)
            pltpu.sync_copy(pw.at[pl.ds(base, CH)], pbuf)
            pltpu.sync_copy(zbuf, accZ.at[ib], add=True)
            pltpu.sync_copy(pbuf, accP.at[ib], add=True)

        return carry

    lax.fori_loop(0, NTRIP, body, 0)
    plsc.subcore_barrier()
    pltpu.sync_copy(accZ.at[pl.ds(rbase, ROWS_PER_TILE)],
                    pz.at[cid, pl.ds(rbase, ROWS_PER_TILE)])
    pltpu.sync_copy(accP.at[pl.ds(rbase, ROWS_PER_TILE)],
                    pp.at[cid, pl.ds(rbase, ROWS_PER_TILE)])


def _run_scatter(rowi, z, pw, zz, zp):
    f32 = jnp.float32
    fn = pl.kernel(
        _scatter_body,
        out_type=[
            jax.ShapeDtypeStruct((NC, N, M), f32),
            jax.ShapeDtypeStruct((NC, N, PW), f32),
        ],
        mesh=plsc.VectorSubcoreMesh(core_axis_name="c", subcore_axis_name="s", num_cores=NC, num_subcores=NS),
        compiler_params=pltpu.CompilerParams(needs_layout_passes=False, use_tc_tiling_on_sc=False),
        scratch_types=[
            pltpu.VMEM_SHARED((N, M), f32), pltpu.VMEM_SHARED((N, PW), f32),
            pltpu.VMEM((CH,), jnp.int32),
            pltpu.VMEM((CH, M), f32), pltpu.VMEM((CH, PW), f32),
        ],
    )
    return fn(rowi, z, pw, zz, zp)


def _update_body(x_ref, hb_ref, pz_ref, pp_ref, mW4r, mb4r, An, Bn,
                 xn_ref, hbn_ref, nA_ref, nB_ref):
    f32 = jnp.float32
    pzs = pz_ref[0] + pz_ref[1]
    pps = pp_ref[0] + pp_ref[1]
    xn_ref[...] = x_ref[...] + pps[:, :3] * (1.0 / NORM)
    deg = pps[:, 3:4]
    hbn = hb_ref[...] + (jnp.dot(pzs, mW4r[...], preferred_element_type=f32)
                         + deg * mb4r[...]) * (1.0 / NORM)
    hbn_ref[...] = hbn
    nA_ref[...] = jnp.dot(hbn, An[...], preferred_element_type=f32)
    nB_ref[...] = jnp.dot(hbn, Bn[...], preferred_element_type=f32)


def _run_update(x, hb, pz, pp, ws):
    f32 = jnp.float32
    nb = 10
    blk = N // nb
    full = lambda s: pl.BlockSpec(s, lambda i: (0,) * len(s))
    return pl.pallas_call(
        _update_body,
        grid=(nb,),
        in_specs=[
            pl.BlockSpec((blk, 3), lambda i: (i, 0)),
            pl.BlockSpec((blk, H), lambda i: (i, 0)),
            pl.BlockSpec((NC, blk, M), lambda i: (0, i, 0)),
            pl.BlockSpec((NC, blk, PW), lambda i: (0, i, 0)),
            full((M, H)), full((1, H)), full((H, M)), full((H, M)),
        ],
        out_specs=[
            pl.BlockSpec((blk, 3), lambda i: (i, 0)),
            pl.BlockSpec((blk, H), lambda i: (i, 0)),
            pl.BlockSpec((blk, M), lambda i: (i, 0)),
            pl.BlockSpec((blk, M), lambda i: (i, 0)),
        ],
        out_shape=[
            jax.ShapeDtypeStruct((N, 3), f32),
            jax.ShapeDtypeStruct((N, H), f32),
            jax.ShapeDtypeStruct((N, M), f32),
            jax.ShapeDtypeStruct((N, M), f32),
        ],
        interpret=_INTERPRET,
    )(x, hb, pz, pp, *ws)


def _final_body(x_ref, hb_ref, pz_ref, pp_ref, mW4r, mb4r, oW1, ob1, tcco_ref,
                xn_ref, ho_ref):
    f32 = jnp.float32
    pzs = pz_ref[0] + pz_ref[1]
    pps = pp_ref[0] + pp_ref[1]
    xn_ref[...] = x_ref[...] + pps[:, :3] * (1.0 / NORM)
    deg = pps[:, 3:4]
    hbn = hb_ref[...] + (jnp.dot(pzs, mW4r[...], preferred_element_type=f32)
                         + deg * mb4r[...]) * (1.0 / NORM)
    ho_ref[...] = (jnp.dot(hbn, oW1[...], preferred_element_type=f32)
                   + ob1[...] - tcco_ref[...])


def _run_final(x, hb, pz, pp, ws, tcco):
    f32 = jnp.float32
    nb = 10
    blk = N // nb
    full = lambda s: pl.BlockSpec(s, lambda i: (0,) * len(s))
    return pl.pallas_call(
        _final_body,
        grid=(nb,),
        in_specs=[
            pl.BlockSpec((blk, 3), lambda i: (i, 0)),
            pl.BlockSpec((blk, H), lambda i: (i, 0)),
            pl.BlockSpec((NC, blk, M), lambda i: (0, i, 0)),
            pl.BlockSpec((NC, blk, PW), lambda i: (0, i, 0)),
            full((M, H)), full((1, H)), full((H, 5)), full((1, 5)),
            pl.BlockSpec((blk, 5), lambda i: (i, 0)),
        ],
        out_specs=[
            pl.BlockSpec((blk, 3), lambda i: (i, 0)),
            pl.BlockSpec((blk, 5), lambda i: (i, 0)),
        ],
        out_shape=[
            jax.ShapeDtypeStruct((N, 3), f32),
            jax.ShapeDtypeStruct((N, 5), f32),
        ],
        interpret=_INTERPRET,
    )(x, hb, pz, pp, *ws, tcco)


def kernel(x, h, c, batch, edge_index, t,
           hW1, hb1, hW2, hb2, dW1, db1, dW2, db2,
           tW1, tb1, tW2, tb2, cW1, cb1, cW2, cb2,
           sW1, sb1, sW2, sb2, oW1, ob1,
           mW1, mb1, mW2, mb2, mW3, mb3, mW4, mb4):
    f32 = jnp.float32
    r1 = lambda v: v.reshape(1, -1)
    row = edge_index[0]
    col = edge_index[1]
    batchf = batch.astype(f32).reshape(N, 1)

    A = [mW1[r][:H] for r in range(3)]
    B = [mW1[r][H:2 * H] for r in range(3)]
    Dd = [mW1[r][2 * H:] for r in range(3)]
    Dfold = [dW2 @ Dd[r] for r in range(3)]
    bfold = [r1(db2 @ Dd[r] + mb1[r]) for r in range(3)]
    S1f = [mW4[r] @ sW1 for r in range(3)]
    sb1f = [r1(mb4[r] @ sW1 + sb1) for r in range(3)]

    hb, nA, nB, tcco = _run_prep(
        h, batchf, t, c,
        (hW1, r1(hb1), hW2, r1(hb2), tW1, r1(tb1), tW2, r1(tb2),
         cW1, r1(cb1), cW2, r1(cb2), oW1, r1(ob1), A[0], B[0]))

    zz = jnp.zeros((N, M), f32)
    zp = jnp.zeros((N, PW), f32)
    sW2r = r1(sW2[:, 0])
    sb2r = sb2.reshape(1, 1)

    for r in range(3):
        x0 = x[:, 0]
        x1 = x[:, 1]
        x2 = x[:, 2]
        gA, gB, dif = _run_gather(nA, nB, x0, x1, x2, row, col)
        z, pw = _run_edge(gA, gB, dif,
                          (r1(dW1[0]), r1(db1), Dfold[r], bfold[r],
                           mW2[r], r1(mb2[r]), mW3[r], r1(mb3[r]),
                           S1f[r], sb1f[r], sW2r, sb2r))
        pz, pp = _run_scatter(row, z, pw, zz, zp)
        if r < 2:
            x, hb, nA, nB = _run_update(
                x, hb, pz, pp, (mW4[r], r1(mb4[r]), A[r + 1], B[r + 1]))
        else:
            x, h_out = _run_final(
                x, hb, pz, pp, (mW4[r], r1(mb4[r]), oW1, r1(ob1)), tcco)
    return (x, h_out)


# speedup vs baseline: 4.3175x; 1.0471x over previous
import functools

import jax
import jax.numpy as jnp
from jax import lax
from jax.experimental import pallas as pl
from jax.experimental.pallas import tpu as pltpu
from jax.experimental.pallas import tpu_sc as plsc

N = 10000
E = 320000
G = 64
H = 128
M = 32
NORM = 100.0

NC = 2
NS = 16
NW = NC * NS
CH = 128
NCHUNK = E // CH
NTRIP = (NCHUNK + NW - 1) // NW
ROWS_PER_TILE = N // NS
ZW = 40

_INTERPRET = False


def _silu(v):
    return v * jax.nn.sigmoid(v)


def _prep_body(h_ref, bf_ref, t_ref, c_ref,
               hW1, hb1, hW2, hb2, tW1, tb1, tW2, tb2, cW1, cb1, cW2, cb2,
               oW1, ob1, A0, B0,
               hb_ref, nA_ref, nB_ref, tcco_ref):
    f32 = jnp.float32
    tcc = (_silu(jnp.dot(t_ref[...], tW1[...], preferred_element_type=f32) + tb1[...])
           @ tW2[...] + tb2[...])
    tcc = tcc + (_silu(jnp.dot(c_ref[...], cW1[...], preferred_element_type=f32) + cb1[...])
                 @ cW2[...] + cb2[...])
    h0 = (_silu(jnp.dot(h_ref[...], hW1[...], preferred_element_type=f32) + hb1[...])
          @ hW2[...] + hb2[...])
    gids = lax.broadcasted_iota(jnp.int32, (h_ref.shape[0], G), 1).astype(f32)
    oh = (bf_ref[...] == gids).astype(f32)
    hb = h0 + jnp.dot(oh, tcc, preferred_element_type=f32)
    hb_ref[...] = hb
    tcco_ref[...] = jnp.dot(oh, jnp.dot(tcc, oW1[...], preferred_element_type=f32),
                            preferred_element_type=f32)
    nA_ref[...] = jnp.dot(hb, A0[...], preferred_element_type=f32)
    nB_ref[...] = jnp.dot(hb, B0[...], preferred_element_type=f32)


def _run_prep(h, batchf, t, c, ws):
    f32 = jnp.float32
    nb = 10
    blk = N // nb
    full = lambda s: pl.BlockSpec(s, lambda i: (0,) * len(s))
    return pl.pallas_call(
        _prep_body,
        grid=(nb,),
        in_specs=[
            pl.BlockSpec((blk, 5), lambda i: (i, 0)),
            pl.BlockSpec((blk, 1), lambda i: (i, 0)),
            full((G, 6)), full((G, 5)),
            full((5, H)), full((1, H)), full((H, H)), full((1, H)),
            full((6, H)), full((1, H)), full((H, H)), full((1, H)),
            full((5, H)), full((1, H)), full((H, H)), full((1, H)),
            full((H, 5)), full((1, 5)), full((H, M)), full((H, M)),
        ],
        out_specs=[
            pl.BlockSpec((blk, H), lambda i: (i, 0)),
            pl.BlockSpec((blk, M), lambda i: (i, 0)),
            pl.BlockSpec((blk, M), lambda i: (i, 0)),
            pl.BlockSpec((blk, 5), lambda i: (i, 0)),
        ],
        out_shape=[
            jax.ShapeDtypeStruct((N, H), f32),
            jax.ShapeDtypeStruct((N, M), f32),
            jax.ShapeDtypeStruct((N, M), f32),
            jax.ShapeDtypeStruct((N, 5), f32),
        ],
        interpret=_INTERPRET,
    )(h, batchf, t, c, *ws)


def _gather_body(nA, nB, x0, x1, x2, rowi, coli,
                 gout,
                 x0v, x1v, x2v, ibr, ibc, bufA, bufB, bufD, sem1, sem2):
    wid = lax.axis_index("s") * NC + lax.axis_index("c")
    pltpu.sync_copy(x0, x0v)
    pltpu.sync_copy(x1, x1v)
    pltpu.sync_copy(x2, x2v)
    iota = lax.iota(jnp.int32, 16)

    def body(j, carry):
        cidx = wid + NW * j

        @pl.when(cidx < NCHUNK)
        def _():
            base = cidx * CH
            pltpu.sync_copy(rowi.at[pl.ds(base, CH)], ibr)
            pltpu.sync_copy(coli.at[pl.ds(base, CH)], ibc)
            cpA = pltpu.async_copy(nA.at[ibr], bufA, sem1)
            cpB = pltpu.async_copy(nB.at[ibc], bufB, sem2)
            for i in range(CH // 16):
                r16 = ibr[pl.ds(i * 16, 16)]
                c16 = ibc[pl.ds(i * 16, 16)]
                d0 = plsc.load_gather(x0v, [r16]) - plsc.load_gather(x0v, [c16])
                d1 = plsc.load_gather(x1v, [r16]) - plsc.load_gather(x1v, [c16])
                d2 = plsc.load_gather(x2v, [r16]) - plsc.load_gather(x2v, [c16])
                sq = d0 * d0 + d1 * d1 + d2 * d2
                er = iota + (i * 16)
                for comp, val in ((0, d0), (1, d1), (2, d2), (3, sq)):
                    cc = jnp.full((16,), comp, jnp.int32)
                    plsc.store_scatter(bufD, [er, cc], val)
            cpA.wait()
            cpB.wait()
            pltpu.sync_copy(bufA, gout.at[0, pl.ds(base, CH)])
            pltpu.sync_copy(bufB, gout.at[1, pl.ds(base, CH)])
            pltpu.sync_copy(bufD, gout.at[2, pl.ds(base, CH)])

        return carry

    lax.fori_loop(0, NTRIP, body, 0)


def _run_gather(nA, nB, x0, x1, x2, rowi, coli):
    f32 = jnp.float32
    fn = pl.kernel(
        _gather_body,
        out_type=[
            jax.ShapeDtypeStruct((3, E, M), f32),
        ],
        mesh=plsc.VectorSubcoreMesh(core_axis_name="c", subcore_axis_name="s", num_cores=NC, num_subcores=NS),
        compiler_params=pltpu.CompilerParams(needs_layout_passes=False, use_tc_tiling_on_sc=False),
        scratch_types=[
            pltpu.VMEM((N,), f32), pltpu.VMEM((N,), f32), pltpu.VMEM((N,), f32),
            pltpu.VMEM((CH,), jnp.int32), pltpu.VMEM((CH,), jnp.int32),
            pltpu.VMEM((CH, M), f32), pltpu.VMEM((CH, M), f32),
            pltpu.VMEM((CH, M), f32),
            pltpu.SemaphoreType.DMA, pltpu.SemaphoreType.DMA,
        ],
    )
    return fn(nA, nB, x0, x1, x2, rowi, coli)


def _edge_body(gout, dW1r, db1r, Dfold, bfold, mW2r, mb2r, mW3r, mb3r,
               S1f, sb1f, sW2r, sb2r, zpw_ref):
    f32 = jnp.float32
    df = gout[2]
    sqd = df[:, 3:4]
    dist = jnp.sqrt(jnp.maximum(sqd, 1e-12))
    d2 = _silu(dist * dW1r[...] + db1r[...])
    pre = (gout[0] + gout[1] + jnp.dot(d2, Dfold[...], preferred_element_type=f32)
           + bfold[...])
    m1 = _silu(pre)
    m2 = _silu(jnp.dot(m1, mW2r[...], preferred_element_type=f32) + mb2r[...])
    z = _silu(jnp.dot(m2, mW3r[...], preferred_element_type=f32) + mb3r[...])
    u = _silu(jnp.dot(z, S1f[...], preferred_element_type=f32) + sb1f[...])
    w = jnp.tanh(jnp.sum(u * sW2r[...], axis=1, keepdims=True) + sb2r[...])
    nrow = df.shape[0]
    zpw_ref[...] = jnp.concatenate(
        [z, df[:, :3] * w, jnp.ones((nrow, 1), f32),
         jnp.zeros((nrow, ZW - M - 4), f32)], axis=1)


def _run_edge(gout, ws):
    f32 = jnp.float32
    nb = 80
    blk = E // nb
    full = lambda s: pl.BlockSpec(s, lambda i: (0,) * len(s))
    return pl.pallas_call(
        _edge_body,
        grid=(nb,),
        in_specs=[
            pl.BlockSpec((3, blk, M), lambda i: (0, i, 0)),
            full((1, M)), full((1, M)), full((M, M)), full((1, M)),
            full((M, M)), full((1, M)), full((M, M)), full((1, M)),
            full((M, H)), full((1, H)), full((1, H)), full((1, 1)),
        ],
        out_specs=[
            pl.BlockSpec((blk, ZW), lambda i: (i, 0)),
        ],
        out_shape=[
            jax.ShapeDtypeStruct((E, ZW), f32),
        ],
        interpret=_INTERPRET,
    )(gout, *ws)


def _scatter_body(rowi, zpw, zz,
                  ps,
                  acc, ib, zbuf):
    cid = lax.axis_index("c")
    sid = lax.axis_index("s")
    wid = sid * NC + cid
    rbase = sid * ROWS_PER_TILE
    pltpu.sync_copy(zz.at[pl.ds(rbase, ROWS_PER_TILE)],
                    acc.at[pl.ds(rbase, ROWS_PER_TILE)])
    plsc.subcore_barrier()

    def body(j, carry):
        cidx = wid + NW * j

        @pl.when(cidx < NCHUNK)
        def _():
            base = cidx * CH
            pltpu.sync_copy(rowi.at[pl.ds(base, CH)], ib)
            pltpu.sync_copy(zpw.at[pl.ds(base, CH)], zbuf)
            pltpu.sync_copy(zbuf, acc.at[ib], add=True)

        return carry

    lax.fori_loop(0, NTRIP, body, 0)
    plsc.subcore_barrier()
    pltpu.sync_copy(acc.at[pl.ds(rbase, ROWS_PER_TILE)],
                    ps.at[cid, pl.ds(rbase, ROWS_PER_TILE)])


def _run_scatter(rowi, zpw, zz):
    f32 = jnp.float32
    fn = pl.kernel(
        _scatter_body,
        out_type=[
            jax.ShapeDtypeStruct((NC, N, ZW), f32),
        ],
        mesh=plsc.VectorSubcoreMesh(core_axis_name="c", subcore_axis_name="s", num_cores=NC, num_subcores=NS),
        compiler_params=pltpu.CompilerParams(needs_layout_passes=False, use_tc_tiling_on_sc=False),
        scratch_types=[
            pltpu.VMEM_SHARED((N, ZW), f32),
            pltpu.VMEM((CH,), jnp.int32),
            pltpu.VMEM((CH, ZW), f32),
        ],
    )
    return fn(rowi, zpw, zz)


def _update_body(x_ref, hb_ref, ps_ref, mW4r, mb4r, An, Bn,
                 xn_ref, hbn_ref, nA_ref, nB_ref):
    f32 = jnp.float32
    ps = ps_ref[0] + ps_ref[1]
    pzs = ps[:, :M]
    xn_ref[...] = x_ref[...] + ps[:, M:M + 3] * (1.0 / NORM)
    deg = ps[:, M + 3:M + 4]
    hbn = hb_ref[...] + (jnp.dot(pzs, mW4r[...], preferred_element_type=f32)
                         + deg * mb4r[...]) * (1.0 / NORM)
    hbn_ref[...] = hbn
    nA_ref[...] = jnp.dot(hbn, An[...], preferred_element_type=f32)
    nB_ref[...] = jnp.dot(hbn, Bn[...], preferred_element_type=f32)


def _run_update(x, hb, ps, ws):
    f32 = jnp.float32
    nb = 10
    blk = N // nb
    full = lambda s: pl.BlockSpec(s, lambda i: (0,) * len(s))
    return pl.pallas_call(
        _update_body,
        grid=(nb,),
        in_specs=[
            pl.BlockSpec((blk, 3), lambda i: (i, 0)),
            pl.BlockSpec((blk, H), lambda i: (i, 0)),
            pl.BlockSpec((NC, blk, ZW), lambda i: (0, i, 0)),
            full((M, H)), full((1, H)), full((H, M)), full((H, M)),
        ],
        out_specs=[
            pl.BlockSpec((blk, 3), lambda i: (i, 0)),
            pl.BlockSpec((blk, H), lambda i: (i, 0)),
            pl.BlockSpec((blk, M), lambda i: (i, 0)),
            pl.BlockSpec((blk, M), lambda i: (i, 0)),
        ],
        out_shape=[
            jax.ShapeDtypeStruct((N, 3), f32),
            jax.ShapeDtypeStruct((N, H), f32),
            jax.ShapeDtypeStruct((N, M), f32),
            jax.ShapeDtypeStruct((N, M), f32),
        ],
        interpret=_INTERPRET,
    )(x, hb, ps, *ws)


def _final_body(x_ref, hb_ref, ps_ref, mW4r, mb4r, oW1, ob1, tcco_ref,
                xn_ref, ho_ref):
    f32 = jnp.float32
    ps = ps_ref[0] + ps_ref[1]
    pzs = ps[:, :M]
    xn_ref[...] = x_ref[...] + ps[:, M:M + 3] * (1.0 / NORM)
    deg = ps[:, M + 3:M + 4]
    hbn = hb_ref[...] + (jnp.dot(pzs, mW4r[...], preferred_element_type=f32)
                         + deg * mb4r[...]) * (1.0 / NORM)
    ho_ref[...] = (jnp.dot(hbn, oW1[...], preferred_element_type=f32)
                   + ob1[...] - tcco_ref[...])


def _run_final(x, hb, ps, ws, tcco):
    f32 = jnp.float32
    nb = 10
    blk = N // nb
    full = lambda s: pl.BlockSpec(s, lambda i: (0,) * len(s))
    return pl.pallas_call(
        _final_body,
        grid=(nb,),
        in_specs=[
            pl.BlockSpec((blk, 3), lambda i: (i, 0)),
            pl.BlockSpec((blk, H), lambda i: (i, 0)),
            pl.BlockSpec((NC, blk, ZW), lambda i: (0, i, 0)),
            full((M, H)), full((1, H)), full((H, 5)), full((1, 5)),
            pl.BlockSpec((blk, 5), lambda i: (i, 0)),
        ],
        out_specs=[
            pl.BlockSpec((blk, 3), lambda i: (i, 0)),
            pl.BlockSpec((blk, 5), lambda i: (i, 0)),
        ],
        out_shape=[
            jax.ShapeDtypeStruct((N, 3), f32),
            jax.ShapeDtypeStruct((N, 5), f32),
        ],
        interpret=_INTERPRET,
    )(x, hb, ps, *ws, tcco)


def kernel(x, h, c, batch, edge_index, t,
           hW1, hb1, hW2, hb2, dW1, db1, dW2, db2,
           tW1, tb1, tW2, tb2, cW1, cb1, cW2, cb2,
           sW1, sb1, sW2, sb2, oW1, ob1,
           mW1, mb1, mW2, mb2, mW3, mb3, mW4, mb4):
    f32 = jnp.float32
    r1 = lambda v: v.reshape(1, -1)
    row = edge_index[0]
    col = edge_index[1]
    batchf = batch.astype(f32).reshape(N, 1)

    A = [mW1[r][:H] for r in range(3)]
    B = [mW1[r][H:2 * H] for r in range(3)]
    Dd = [mW1[r][2 * H:] for r in range(3)]
    Dfold = [dW2 @ Dd[r] for r in range(3)]
    bfold = [r1(db2 @ Dd[r] + mb1[r]) for r in range(3)]
    S1f = [mW4[r] @ sW1 for r in range(3)]
    sb1f = [r1(mb4[r] @ sW1 + sb1) for r in range(3)]

    hb, nA, nB, tcco = _run_prep(
        h, batchf, t, c,
        (hW1, r1(hb1), hW2, r1(hb2), tW1, r1(tb1), tW2, r1(tb2),
         cW1, r1(cb1), cW2, r1(cb2), oW1, r1(ob1), A[0], B[0]))

    zz = jnp.zeros((N, ZW), f32)
    sW2r = r1(sW2[:, 0])
    sb2r = sb2.reshape(1, 1)

    for r in range(3):
        x0 = x[:, 0]
        x1 = x[:, 1]
        x2 = x[:, 2]
        gout, = _run_gather(nA, nB, x0, x1, x2, row, col)
        zpw, = _run_edge(gout,
                         (r1(dW1[0]), r1(db1), Dfold[r], bfold[r],
                          mW2[r], r1(mb2[r]), mW3[r], r1(mb3[r]),
                          S1f[r], sb1f[r], sW2r, sb2r))
        ps, = _run_scatter(row, zpw, zz)
        if r < 2:
            x, hb, nA, nB = _run_update(
                x, hb, ps, (mW4[r], r1(mb4[r]), A[r + 1], B[r + 1]))
        else:
            x, h_out = _run_final(
                x, hb, ps, (mW4[r], r1(mb4[r]), oW1, r1(ob1)), tcco)
    return (x, h_out)


# speedup vs baseline: 9.9053x; 2.2942x over previous
import functools

import jax
import jax.numpy as jnp
from jax import lax
from jax.experimental import pallas as pl
from jax.experimental.pallas import tpu as pltpu
from jax.experimental.pallas import tpu_sc as plsc

N = 10000
E = 320000
G = 64
H = 128
M = 32
NORM = 100.0

NC = 2
NS = 16
NW = NC * NS
CH = 128
NCHUNK = E // CH
NTRIP = (NCHUNK + NW - 1) // NW
ROWS_PER_TILE = N // NS
EP = E * M // 128

_INTERPRET = False


def _silu(v):
    return v * jax.nn.sigmoid(v)


def _prep_body(h_ref, bf_ref, t_ref, c_ref,
               hW1, hb1, hW2, hb2, tW1, tb1, tW2, tb2, cW1, cb1, cW2, cb2,
               oW1, ob1, A0, B0,
               hb_ref, nA_ref, nB_ref, tcco_ref):
    f32 = jnp.float32
    tcc = (_silu(jnp.dot(t_ref[...], tW1[...], preferred_element_type=f32) + tb1[...])
           @ tW2[...] + tb2[...])
    tcc = tcc + (_silu(jnp.dot(c_ref[...], cW1[...], preferred_element_type=f32) + cb1[...])
                 @ cW2[...] + cb2[...])
    h0 = (_silu(jnp.dot(h_ref[...], hW1[...], preferred_element_type=f32) + hb1[...])
          @ hW2[...] + hb2[...])
    gids = lax.broadcasted_iota(jnp.int32, (h_ref.shape[0], G), 1).astype(f32)
    oh = (bf_ref[...] == gids).astype(f32)
    hb = h0 + jnp.dot(oh, tcc, preferred_element_type=f32)
    hb_ref[...] = hb
    tcco_ref[...] = jnp.dot(oh, jnp.dot(tcc, oW1[...], preferred_element_type=f32),
                            preferred_element_type=f32)
    nA_ref[...] = jnp.dot(hb, A0[...], preferred_element_type=f32)
    nB_ref[...] = jnp.dot(hb, B0[...], preferred_element_type=f32)


def _run_prep(h, batchf, t, c, ws):
    f32 = jnp.float32
    nb = 10
    blk = N // nb
    full = lambda s: pl.BlockSpec(s, lambda i: (0,) * len(s))
    return pl.pallas_call(
        _prep_body,
        grid=(nb,),
        in_specs=[
            pl.BlockSpec((blk, 5), lambda i: (i, 0)),
            pl.BlockSpec((blk, 1), lambda i: (i, 0)),
            full((G, 6)), full((G, 5)),
            full((5, H)), full((1, H)), full((H, H)), full((1, H)),
            full((6, H)), full((1, H)), full((H, H)), full((1, H)),
            full((5, H)), full((1, H)), full((H, H)), full((1, H)),
            full((H, 5)), full((1, 5)), full((H, M)), full((H, M)),
        ],
        out_specs=[
            pl.BlockSpec((blk, H), lambda i: (i, 0)),
            pl.BlockSpec((blk, M), lambda i: (i, 0)),
            pl.BlockSpec((blk, M), lambda i: (i, 0)),
            pl.BlockSpec((blk, 5), lambda i: (i, 0)),
        ],
        out_shape=[
            jax.ShapeDtypeStruct((N, H), f32),
            jax.ShapeDtypeStruct((N, M), f32),
            jax.ShapeDtypeStruct((N, M), f32),
            jax.ShapeDtypeStruct((N, 5), f32),
        ],
        interpret=_INTERPRET,
    )(h, batchf, t, c, *ws)


def _gather_body(nA, nB, x0, x1, x2, rowi, coli,
                 gA, gB, dif,
                 x0v, x1v, x2v, ibr, ibc, bufA, bufB, bufD, sem1, sem2):
    wid = lax.axis_index("s") * NC + lax.axis_index("c")
    pltpu.sync_copy(x0, x0v)
    pltpu.sync_copy(x1, x1v)
    pltpu.sync_copy(x2, x2v)
    iota = lax.iota(jnp.int32, 16)

    z16 = jnp.zeros((16,), jnp.float32)

    def zbody(j, carry):
        idx = j * 16 + iota
        plsc.store_scatter(bufD, [idx >> 5, idx & 31], z16)
        return carry

    lax.fori_loop(0, (CH * M) // 16, zbody, 0)

    def body(j, carry):
        cidx = wid + NW * j

        @pl.when(cidx < NCHUNK)
        def _():
            base = cidx * CH
            pltpu.sync_copy(rowi.at[pl.ds(base, CH)], ibr)
            pltpu.sync_copy(coli.at[pl.ds(base, CH)], ibc)
            cpA = pltpu.async_copy(nA.at[ibr], bufA, sem1)
            cpB = pltpu.async_copy(nB.at[ibc], bufB, sem2)
            for i in range(CH // 16):
                r16 = ibr[pl.ds(i * 16, 16)]
                c16 = ibc[pl.ds(i * 16, 16)]
                d0 = plsc.load_gather(x0v, [r16]) - plsc.load_gather(x0v, [c16])
                d1 = plsc.load_gather(x1v, [r16]) - plsc.load_gather(x1v, [c16])
                d2 = plsc.load_gather(x2v, [r16]) - plsc.load_gather(x2v, [c16])
                sq = d0 * d0 + d1 * d1 + d2 * d2
                er = iota + (i * 16)
                for comp, val in ((0, d0), (1, d1), (2, d2), (3, sq)):
                    cc = jnp.full((16,), comp, jnp.int32)
                    plsc.store_scatter(bufD, [er, cc], val)
            cpA.wait()
            cpB.wait()
            pltpu.sync_copy(bufA, gA.at[pl.ds(base, CH)])
            pltpu.sync_copy(bufB, gB.at[pl.ds(base, CH)])
            pltpu.sync_copy(bufD, dif.at[pl.ds(base, CH)])

        return carry

    lax.fori_loop(0, NTRIP, body, 0)


def _run_gather(nA, nB, x0, x1, x2, rowi, coli):
    f32 = jnp.float32
    fn = pl.kernel(
        _gather_body,
        out_type=[
            jax.ShapeDtypeStruct((E, M), f32),
            jax.ShapeDtypeStruct((E, M), f32),
            jax.ShapeDtypeStruct((E, M), f32),
        ],
        mesh=plsc.VectorSubcoreMesh(core_axis_name="c", subcore_axis_name="s", num_cores=NC, num_subcores=NS),
        compiler_params=pltpu.CompilerParams(needs_layout_passes=False, use_tc_tiling_on_sc=False),
        scratch_types=[
            pltpu.VMEM((N,), f32), pltpu.VMEM((N,), f32), pltpu.VMEM((N,), f32),
            pltpu.VMEM((CH,), jnp.int32), pltpu.VMEM((CH,), jnp.int32),
            pltpu.VMEM((CH, M), f32), pltpu.VMEM((CH, M), f32),
            pltpu.VMEM((CH, M), f32),
            pltpu.SemaphoreType.DMA, pltpu.SemaphoreType.DMA,
        ],
    )
    return fn(nA, nB, x0, x1, x2, rowi, coli)


def _edge_body(gA, gB, dfp, SEL3, dW1r4, db1r4, Dbd, bf4, W2bd, b24, W3bd, b34,
               S1f0, S1f1, S1f2, S1f3, sb1f, sW2r, sb2r, BC4, zw_ref):
    f32 = jnp.float32
    df = dfp[...]
    sqb = jnp.dot(df, SEL3[...], preferred_element_type=f32)
    dist = jnp.sqrt(jnp.maximum(sqb, 1e-12))
    d2 = _silu(dist * dW1r4[...] + db1r4[...])
    pre = (gA[...] + gB[...] + jnp.dot(d2, Dbd[...], preferred_element_type=f32)
           + bf4[...])
    m1 = _silu(pre)
    m2 = _silu(jnp.dot(m1, W2bd[...], preferred_element_type=f32) + b24[...])
    z = _silu(jnp.dot(m2, W3bd[...], preferred_element_type=f32) + b34[...])
    s_cols = []
    for Sj in (S1f0, S1f1, S1f2, S1f3):
        uj = _silu(jnp.dot(z, Sj[...], preferred_element_type=f32) + sb1f[...])
        s_cols.append(jnp.sum(uj * sW2r[...], axis=1, keepdims=True))
    w4 = jnp.tanh(jnp.concatenate(s_cols, axis=1) + sb2r[...])
    wb = jnp.dot(w4, BC4[...], preferred_element_type=f32)
    zw_ref[0] = z
    zw_ref[1] = df * wb


def _run_edge(gA, gB, dfp, ws):
    f32 = jnp.float32
    nb = 80
    blk = EP // nb
    full = lambda s: pl.BlockSpec(s, lambda i: (0,) * len(s))
    return pl.pallas_call(
        _edge_body,
        grid=(nb,),
        in_specs=[
            pl.BlockSpec((blk, 128), lambda i: (i, 0)),
            pl.BlockSpec((blk, 128), lambda i: (i, 0)),
            pl.BlockSpec((blk, 128), lambda i: (i, 0)),
            full((128, 128)), full((1, 128)), full((1, 128)),
            full((128, 128)), full((1, 128)),
            full((128, 128)), full((1, 128)), full((128, 128)), full((1, 128)),
            full((128, H)), full((128, H)), full((128, H)), full((128, H)),
            full((1, H)), full((1, H)), full((1, 1)),
            full((4, 128)),
        ],
        out_specs=[
            pl.BlockSpec((2, blk, 128), lambda i: (0, i, 0)),
        ],
        out_shape=[
            jax.ShapeDtypeStruct((2, EP, 128), f32),
        ],
        interpret=_INTERPRET,
    )(gA, gB, dfp, *ws)


def _scatter_body(rowi, zw, zz,
                  ps,
                  accZ, accW, ib, zbuf, wbuf):
    cid = lax.axis_index("c")
    sid = lax.axis_index("s")
    wid = sid * NC + cid
    rbase = sid * ROWS_PER_TILE
    pltpu.sync_copy(zz.at[pl.ds(rbase, ROWS_PER_TILE)],
                    accZ.at[pl.ds(rbase, ROWS_PER_TILE)])
    pltpu.sync_copy(zz.at[pl.ds(rbase, ROWS_PER_TILE)],
                    accW.at[pl.ds(rbase, ROWS_PER_TILE)])
    plsc.subcore_barrier()

    def body(j, carry):
        cidx = wid + NW * j

        @pl.when(cidx < NCHUNK)
        def _():
            base = cidx * CH
            pltpu.sync_copy(rowi.at[pl.ds(base, CH)], ib)
            pltpu.sync_copy(zw.at[0, pl.ds(base, CH)], zbuf)
            pltpu.sync_copy(zw.at[1, pl.ds(base, CH)], wbuf)
            pltpu.sync_copy(zbuf, accZ.at[ib], add=True)
            pltpu.sync_copy(wbuf, accW.at[ib], add=True)

        return carry

    lax.fori_loop(0, NTRIP, body, 0)
    plsc.subcore_barrier()
    pltpu.sync_copy(accZ.at[pl.ds(rbase, ROWS_PER_TILE)],
                    ps.at[cid, 0, pl.ds(rbase, ROWS_PER_TILE)])
    pltpu.sync_copy(accW.at[pl.ds(rbase, ROWS_PER_TILE)],
                    ps.at[cid, 1, pl.ds(rbase, ROWS_PER_TILE)])


def _run_scatter(rowi, zw, zz):
    f32 = jnp.float32
    fn = pl.kernel(
        _scatter_body,
        out_type=[
            jax.ShapeDtypeStruct((NC, 2, N, M), f32),
        ],
        mesh=plsc.VectorSubcoreMesh(core_axis_name="c", subcore_axis_name="s", num_cores=NC, num_subcores=NS),
        compiler_params=pltpu.CompilerParams(needs_layout_passes=False, use_tc_tiling_on_sc=False),
        scratch_types=[
            pltpu.VMEM_SHARED((N, M), f32), pltpu.VMEM_SHARED((N, M), f32),
            pltpu.VMEM((CH,), jnp.int32),
            pltpu.VMEM((CH, M), f32), pltpu.VMEM((CH, M), f32),
        ],
    )
    return fn(rowi, zw, zz)


def _deg_body(rowi, ones8, zz8,
              dg,
              acc, ib, ob):
    cid = lax.axis_index("c")
    sid = lax.axis_index("s")
    wid = sid * NC + cid
    rbase = sid * ROWS_PER_TILE
    pltpu.sync_copy(ones8, ob)
    pltpu.sync_copy(zz8.at[pl.ds(rbase, ROWS_PER_TILE)],
                    acc.at[pl.ds(rbase, ROWS_PER_TILE)])
    plsc.subcore_barrier()

    def body(j, carry):
        cidx = wid + NW * j

        @pl.when(cidx < NCHUNK)
        def _():
            base = cidx * CH
            pltpu.sync_copy(rowi.at[pl.ds(base, CH)], ib)
            pltpu.sync_copy(ob, acc.at[ib], add=True)

        return carry

    lax.fori_loop(0, NTRIP, body, 0)
    plsc.subcore_barrier()
    pltpu.sync_copy(acc.at[pl.ds(rbase, ROWS_PER_TILE)],
                    dg.at[cid, pl.ds(rbase, ROWS_PER_TILE)])


def _run_deg(rowi, ones8, zz8):
    f32 = jnp.float32
    fn = pl.kernel(
        _deg_body,
        out_type=[
            jax.ShapeDtypeStruct((NC, N, 8), f32),
        ],
        mesh=plsc.VectorSubcoreMesh(core_axis_name="c", subcore_axis_name="s", num_cores=NC, num_subcores=NS),
        compiler_params=pltpu.CompilerParams(needs_layout_passes=False, use_tc_tiling_on_sc=False),
        scratch_types=[
            pltpu.VMEM_SHARED((N, 8), f32),
            pltpu.VMEM((CH,), jnp.int32),
            pltpu.VMEM((CH, 8), f32),
        ],
    )
    return fn(rowi, ones8, zz8)


def _update_body(x_ref, hb_ref, ps_ref, dg_ref, mW4r, mb4r, An, Bn,
                 xn_ref, hbn_ref, nA_ref, nB_ref):
    f32 = jnp.float32
    ps = ps_ref[0] + ps_ref[1]
    pzs = ps[0]
    xn_ref[...] = x_ref[...] + ps[1][:, :3] * (1.0 / NORM)
    deg = dg_ref[0][:, 0:1] + dg_ref[1][:, 0:1]
    hbn = hb_ref[...] + (jnp.dot(pzs, mW4r[...], preferred_element_type=f32)
                         + deg * mb4r[...]) * (1.0 / NORM)
    hbn_ref[...] = hbn
    nA_ref[...] = jnp.dot(hbn, An[...], preferred_element_type=f32)
    nB_ref[...] = jnp.dot(hbn, Bn[...], preferred_element_type=f32)


def _run_update(x, hb, ps, dg, ws):
    f32 = jnp.float32
    nb = 10
    blk = N // nb
    full = lambda s: pl.BlockSpec(s, lambda i: (0,) * len(s))
    return pl.pallas_call(
        _update_body,
        grid=(nb,),
        in_specs=[
            pl.BlockSpec((blk, 3), lambda i: (i, 0)),
            pl.BlockSpec((blk, H), lambda i: (i, 0)),
            pl.BlockSpec((NC, 2, blk, M), lambda i: (0, 0, i, 0)),
            pl.BlockSpec((NC, blk, 8), lambda i: (0, i, 0)),
            full((M, H)), full((1, H)), full((H, M)), full((H, M)),
        ],
        out_specs=[
            pl.BlockSpec((blk, 3), lambda i: (i, 0)),
            pl.BlockSpec((blk, H), lambda i: (i, 0)),
            pl.BlockSpec((blk, M), lambda i: (i, 0)),
            pl.BlockSpec((blk, M), lambda i: (i, 0)),
        ],
        out_shape=[
            jax.ShapeDtypeStruct((N, 3), f32),
            jax.ShapeDtypeStruct((N, H), f32),
            jax.ShapeDtypeStruct((N, M), f32),
            jax.ShapeDtypeStruct((N, M), f32),
        ],
        interpret=_INTERPRET,
    )(x, hb, ps, dg, *ws)


def _final_body(x_ref, hb_ref, ps_ref, dg_ref, mW4r, mb4r, oW1, ob1, tcco_ref,
                xn_ref, ho_ref):
    f32 = jnp.float32
    ps = ps_ref[0] + ps_ref[1]
    pzs = ps[0]
    xn_ref[...] = x_ref[...] + ps[1][:, :3] * (1.0 / NORM)
    deg = dg_ref[0][:, 0:1] + dg_ref[1][:, 0:1]
    hbn = hb_ref[...] + (jnp.dot(pzs, mW4r[...], preferred_element_type=f32)
                         + deg * mb4r[...]) * (1.0 / NORM)
    ho_ref[...] = (jnp.dot(hbn, oW1[...], preferred_element_type=f32)
                   + ob1[...] - tcco_ref[...])


def _run_final(x, hb, ps, dg, ws, tcco):
    f32 = jnp.float32
    nb = 10
    blk = N // nb
    full = lambda s: pl.BlockSpec(s, lambda i: (0,) * len(s))
    return pl.pallas_call(
        _final_body,
        grid=(nb,),
        in_specs=[
            pl.BlockSpec((blk, 3), lambda i: (i, 0)),
            pl.BlockSpec((blk, H), lambda i: (i, 0)),
            pl.BlockSpec((NC, 2, blk, M), lambda i: (0, 0, i, 0)),
            pl.BlockSpec((NC, blk, 8), lambda i: (0, i, 0)),
            full((M, H)), full((1, H)), full((H, 5)), full((1, 5)),
            pl.BlockSpec((blk, 5), lambda i: (i, 0)),
        ],
        out_specs=[
            pl.BlockSpec((blk, 3), lambda i: (i, 0)),
            pl.BlockSpec((blk, 5), lambda i: (i, 0)),
        ],
        out_shape=[
            jax.ShapeDtypeStruct((N, 3), f32),
            jax.ShapeDtypeStruct((N, 5), f32),
        ],
        interpret=_INTERPRET,
    )(x, hb, ps, dg, *ws, tcco)


def kernel(x, h, c, batch, edge_index, t,
           hW1, hb1, hW2, hb2, dW1, db1, dW2, db2,
           tW1, tb1, tW2, tb2, cW1, cb1, cW2, cb2,
           sW1, sb1, sW2, sb2, oW1, ob1,
           mW1, mb1, mW2, mb2, mW3, mb3, mW4, mb4):
    f32 = jnp.float32
    r1 = lambda v: v.reshape(1, -1)
    row = edge_index[0]
    col = edge_index[1]
    batchf = batch.astype(f32).reshape(N, 1)

    A = [mW1[r][:H] for r in range(3)]
    B = [mW1[r][H:2 * H] for r in range(3)]
    Dd = [mW1[r][2 * H:] for r in range(3)]
    Dfold = [dW2 @ Dd[r] for r in range(3)]
    bfold = [r1(db2 @ Dd[r] + mb1[r]) for r in range(3)]
    S1f = [mW4[r] @ sW1 for r in range(3)]
    sb1f = [r1(mb4[r] @ sW1 + sb1) for r in range(3)]

    eye4 = jnp.eye(4, dtype=f32)
    bd = lambda W: jnp.kron(eye4, W)
    t4 = lambda b: jnp.tile(b, (1, 4))
    sel3 = jnp.zeros((128, 128), f32)
    bc4 = jnp.zeros((4, 128), f32)
    for j in range(4):
        sel3 = sel3.at[32 * j + 3, 32 * j:32 * j + 32].set(1.0)
        bc4 = bc4.at[j, 32 * j:32 * j + 32].set(1.0)

    hb, nA, nB, tcco = _run_prep(
        h, batchf, t, c,
        (hW1, r1(hb1), hW2, r1(hb2), tW1, r1(tb1), tW2, r1(tb2),
         cW1, r1(cb1), cW2, r1(cb2), oW1, r1(ob1), A[0], B[0]))

    zz = jnp.zeros((N, M), f32)
    sW2r = r1(sW2[:, 0])
    sb2r = sb2.reshape(1, 1)
    dW1r4 = t4(r1(dW1[0]))
    db1r4 = t4(r1(db1))

    dg, = _run_deg(row, jnp.ones((CH, 8), f32), jnp.zeros((N, 8), f32))

    for r in range(3):
        x0 = x[:, 0]
        x1 = x[:, 1]
        x2 = x[:, 2]
        gA, gB, dif = _run_gather(nA, nB, x0, x1, x2, row, col)
        S1fj = [jnp.zeros((128, H), f32).at[32 * j:32 * j + 32].set(S1f[r])
                for j in range(4)]
        zwp, = _run_edge(gA.reshape(EP, 128), gB.reshape(EP, 128),
                         dif.reshape(EP, 128),
                         (sel3, dW1r4, db1r4, bd(Dfold[r]), t4(bfold[r]),
                          bd(mW2[r]), t4(r1(mb2[r])), bd(mW3[r]), t4(r1(mb3[r])),
                          *S1fj, sb1f[r], sW2r, sb2r, bc4))
        ps, = _run_scatter(row, zwp.reshape(2, E, M), zz)
        if r < 2:
            x, hb, nA, nB = _run_update(
                x, hb, ps, dg, (mW4[r], r1(mb4[r]), A[r + 1], B[r + 1]))
        else:
            x, h_out = _run_final(
                x, hb, ps, dg, (mW4[r], r1(mb4[r]), oW1, r1(ob1)), tcco)
    return (x, h_out)


# speedup vs baseline: 12.4080x; 1.2527x over previous
import functools

import jax
import jax.numpy as jnp
from jax import lax
from jax.experimental import pallas as pl
from jax.experimental.pallas import tpu as pltpu
from jax.experimental.pallas import tpu_sc as plsc

N = 10000
E = 320000
G = 64
H = 128
M = 32
NORM = 100.0

NC = 2
NS = 16
NW = NC * NS
CH = 256
NCHUNK = E // CH
NTRIP = (NCHUNK + NW - 1) // NW
ROWS_PER_TILE = N // NS
EP = E * M // 128

_INTERPRET = False


def _silu(v):
    return v * jax.nn.sigmoid(v)


def _prep_body(h_ref, bf_ref, t_ref, c_ref,
               hW1, hb1, hW2, hb2, tW1, tb1, tW2, tb2, cW1, cb1, cW2, cb2,
               oW1, ob1, A0, B0,
               hb_ref, nA_ref, nB_ref, tcco_ref):
    f32 = jnp.float32
    tcc = (_silu(jnp.dot(t_ref[...], tW1[...], preferred_element_type=f32) + tb1[...])
           @ tW2[...] + tb2[...])
    tcc = tcc + (_silu(jnp.dot(c_ref[...], cW1[...], preferred_element_type=f32) + cb1[...])
                 @ cW2[...] + cb2[...])
    h0 = (_silu(jnp.dot(h_ref[...], hW1[...], preferred_element_type=f32) + hb1[...])
          @ hW2[...] + hb2[...])
    gids = lax.broadcasted_iota(jnp.int32, (h_ref.shape[0], G), 1).astype(f32)
    oh = (bf_ref[...] == gids).astype(f32)
    hb = h0 + jnp.dot(oh, tcc, preferred_element_type=f32)
    hb_ref[...] = hb
    tcco_ref[...] = jnp.dot(oh, jnp.dot(tcc, oW1[...], preferred_element_type=f32),
                            preferred_element_type=f32)
    nA_ref[...] = jnp.dot(hb, A0[...], preferred_element_type=f32)
    nB_ref[...] = jnp.dot(hb, B0[...], preferred_element_type=f32)


def _run_prep(h, batchf, t, c, ws):
    f32 = jnp.float32
    nb = 10
    blk = N // nb
    full = lambda s: pl.BlockSpec(s, lambda i: (0,) * len(s))
    return pl.pallas_call(
        _prep_body,
        grid=(nb,),
        in_specs=[
            pl.BlockSpec((blk, 5), lambda i: (i, 0)),
            pl.BlockSpec((blk, 1), lambda i: (i, 0)),
            full((G, 6)), full((G, 5)),
            full((5, H)), full((1, H)), full((H, H)), full((1, H)),
            full((6, H)), full((1, H)), full((H, H)), full((1, H)),
            full((5, H)), full((1, H)), full((H, H)), full((1, H)),
            full((H, 5)), full((1, 5)), full((H, M)), full((H, M)),
        ],
        out_specs=[
            pl.BlockSpec((blk, H), lambda i: (i, 0)),
            pl.BlockSpec((blk, M), lambda i: (i, 0)),
            pl.BlockSpec((blk, M), lambda i: (i, 0)),
            pl.BlockSpec((blk, 5), lambda i: (i, 0)),
        ],
        out_shape=[
            jax.ShapeDtypeStruct((N, H), f32),
            jax.ShapeDtypeStruct((N, M), f32),
            jax.ShapeDtypeStruct((N, M), f32),
            jax.ShapeDtypeStruct((N, 5), f32),
        ],
        interpret=_INTERPRET,
    )(h, batchf, t, c, *ws)


def _gather_body(nA, nB, x0, x1, x2, rowi, coli,
                 gA, gB, dif,
                 x0v, x1v, x2v, ibr, ibc, bufA, bufB, bufD, sem1, sem2):
    wid = lax.axis_index("s") * NC + lax.axis_index("c")
    pltpu.sync_copy(x0, x0v)
    pltpu.sync_copy(x1, x1v)
    pltpu.sync_copy(x2, x2v)
    iota = lax.iota(jnp.int32, 16)

    z16 = jnp.zeros((16,), jnp.float32)

    def zbody(j, carry):
        idx = j * 16 + iota
        plsc.store_scatter(bufD, [idx >> 5, idx & 31], z16)
        return carry

    lax.fori_loop(0, (CH * M) // 16, zbody, 0)

    def body(j, carry):
        cidx = wid + NW * j

        @pl.when(cidx < NCHUNK)
        def _():
            base = cidx * CH
            pltpu.sync_copy(rowi.at[pl.ds(base, CH)], ibr)
            pltpu.sync_copy(coli.at[pl.ds(base, CH)], ibc)
            cpA = pltpu.async_copy(nA.at[ibr], bufA, sem1)
            cpB = pltpu.async_copy(nB.at[ibc], bufB, sem2)
            for i in range(CH // 16):
                r16 = ibr[pl.ds(i * 16, 16)]
                c16 = ibc[pl.ds(i * 16, 16)]
                d0 = plsc.load_gather(x0v, [r16]) - plsc.load_gather(x0v, [c16])
                d1 = plsc.load_gather(x1v, [r16]) - plsc.load_gather(x1v, [c16])
                d2 = plsc.load_gather(x2v, [r16]) - plsc.load_gather(x2v, [c16])
                sq = d0 * d0 + d1 * d1 + d2 * d2
                er = iota + (i * 16)
                for comp, val in ((0, d0), (1, d1), (2, d2), (3, sq)):
                    cc = jnp.full((16,), comp, jnp.int32)
                    plsc.store_scatter(bufD, [er, cc], val)
            cpA.wait()
            cpB.wait()
            pltpu.sync_copy(bufA, gA.at[pl.ds(base, CH)])
            pltpu.sync_copy(bufB, gB.at[pl.ds(base, CH)])
            pltpu.sync_copy(bufD, dif.at[pl.ds(base, CH)])

        return carry

    lax.fori_loop(0, NTRIP, body, 0)


def _run_gather(nA, nB, x0, x1, x2, rowi, coli):
    f32 = jnp.float32
    fn = pl.kernel(
        _gather_body,
        out_type=[
            jax.ShapeDtypeStruct((E, M), f32),
            jax.ShapeDtypeStruct((E, M), f32),
            jax.ShapeDtypeStruct((E, M), f32),
        ],
        mesh=plsc.VectorSubcoreMesh(core_axis_name="c", subcore_axis_name="s", num_cores=NC, num_subcores=NS),
        compiler_params=pltpu.CompilerParams(needs_layout_passes=False, use_tc_tiling_on_sc=False),
        scratch_types=[
            pltpu.VMEM((N,), f32), pltpu.VMEM((N,), f32), pltpu.VMEM((N,), f32),
            pltpu.VMEM((CH,), jnp.int32), pltpu.VMEM((CH,), jnp.int32),
            pltpu.VMEM((CH, M), f32), pltpu.VMEM((CH, M), f32),
            pltpu.VMEM((CH, M), f32),
            pltpu.SemaphoreType.DMA, pltpu.SemaphoreType.DMA,
        ],
    )
    return fn(nA, nB, x0, x1, x2, rowi, coli)


def _edge_body(gA, gB, dfp, SEL3, dW1r4, db1r4, Dbd, bf4, W2bd, b24, W3bd, b34,
               S1f0, S1f1, S1f2, S1f3, sb1f, sW2r, sb2r, BC4, zw_ref):
    f32 = jnp.float32
    df = dfp[...]
    sqb = jnp.dot(df, SEL3[...], preferred_element_type=f32)
    dist = jnp.sqrt(jnp.maximum(sqb, 1e-12))
    d2 = _silu(dist * dW1r4[...] + db1r4[...])
    pre = (gA[...] + gB[...] + jnp.dot(d2, Dbd[...], preferred_element_type=f32)
           + bf4[...])
    m1 = _silu(pre)
    m2 = _silu(jnp.dot(m1, W2bd[...], preferred_element_type=f32) + b24[...])
    z = _silu(jnp.dot(m2, W3bd[...], preferred_element_type=f32) + b34[...])
    s_cols = []
    for Sj in (S1f0, S1f1, S1f2, S1f3):
        uj = _silu(jnp.dot(z, Sj[...], preferred_element_type=f32) + sb1f[...])
        s_cols.append(jnp.sum(uj * sW2r[...], axis=1, keepdims=True))
    w4 = jnp.tanh(jnp.concatenate(s_cols, axis=1) + sb2r[...])
    wb = jnp.dot(w4, BC4[...], preferred_element_type=f32)
    zw_ref[0] = z
    zw_ref[1] = df * wb


def _run_edge(gA, gB, dfp, ws):
    f32 = jnp.float32
    nb = 80
    blk = EP // nb
    full = lambda s: pl.BlockSpec(s, lambda i: (0,) * len(s))
    return pl.pallas_call(
        _edge_body,
        grid=(nb,),
        in_specs=[
            pl.BlockSpec((blk, 128), lambda i: (i, 0)),
            pl.BlockSpec((blk, 128), lambda i: (i, 0)),
            pl.BlockSpec((blk, 128), lambda i: (i, 0)),
            full((128, 128)), full((1, 128)), full((1, 128)),
            full((128, 128)), full((1, 128)),
            full((128, 128)), full((1, 128)), full((128, 128)), full((1, 128)),
            full((128, H)), full((128, H)), full((128, H)), full((128, H)),
            full((1, H)), full((1, H)), full((1, 1)),
            full((4, 128)),
        ],
        out_specs=[
            pl.BlockSpec((2, blk, 128), lambda i: (0, i, 0)),
        ],
        out_shape=[
            jax.ShapeDtypeStruct((2, EP, 128), f32),
        ],
        interpret=_INTERPRET,
    )(gA, gB, dfp, *ws)


def _scatter_body(rowi, zw, zz,
                  ps,
                  accZ, accW, ib, zbuf, wbuf):
    cid = lax.axis_index("c")
    sid = lax.axis_index("s")
    wid = sid * NC + cid
    rbase = sid * ROWS_PER_TILE
    pltpu.sync_copy(zz.at[pl.ds(rbase, ROWS_PER_TILE)],
                    accZ.at[pl.ds(rbase, ROWS_PER_TILE)])
    pltpu.sync_copy(zz.at[pl.ds(rbase, ROWS_PER_TILE)],
                    accW.at[pl.ds(rbase, ROWS_PER_TILE)])
    plsc.subcore_barrier()

    def body(j, carry):
        cidx = wid + NW * j

        @pl.when(cidx < NCHUNK)
        def _():
            base = cidx * CH
            pltpu.sync_copy(rowi.at[pl.ds(base, CH)], ib)
            pltpu.sync_copy(zw.at[0, pl.ds(base, CH)], zbuf)
            pltpu.sync_copy(zw.at[1, pl.ds(base, CH)], wbuf)
            pltpu.sync_copy(zbuf, accZ.at[ib], add=True)
            pltpu.sync_copy(wbuf, accW.at[ib], add=True)

        return carry

    lax.fori_loop(0, NTRIP, body, 0)
    plsc.subcore_barrier()
    pltpu.sync_copy(accZ.at[pl.ds(rbase, ROWS_PER_TILE)],
                    ps.at[cid, 0, pl.ds(rbase, ROWS_PER_TILE)])
    pltpu.sync_copy(accW.at[pl.ds(rbase, ROWS_PER_TILE)],
                    ps.at[cid, 1, pl.ds(rbase, ROWS_PER_TILE)])


def _run_scatter(rowi, zw, zz):
    f32 = jnp.float32
    fn = pl.kernel(
        _scatter_body,
        out_type=[
            jax.ShapeDtypeStruct((NC, 2, N, M), f32),
        ],
        mesh=plsc.VectorSubcoreMesh(core_axis_name="c", subcore_axis_name="s", num_cores=NC, num_subcores=NS),
        compiler_params=pltpu.CompilerParams(needs_layout_passes=False, use_tc_tiling_on_sc=False),
        scratch_types=[
            pltpu.VMEM_SHARED((N, M), f32), pltpu.VMEM_SHARED((N, M), f32),
            pltpu.VMEM((CH,), jnp.int32),
            pltpu.VMEM((CH, M), f32), pltpu.VMEM((CH, M), f32),
        ],
    )
    return fn(rowi, zw, zz)


def _deg_body(rowi, ones8, zz8,
              dg,
              acc, ib, ob):
    cid = lax.axis_index("c")
    sid = lax.axis_index("s")
    wid = sid * NC + cid
    rbase = sid * ROWS_PER_TILE
    pltpu.sync_copy(ones8, ob)
    pltpu.sync_copy(zz8.at[pl.ds(rbase, ROWS_PER_TILE)],
                    acc.at[pl.ds(rbase, ROWS_PER_TILE)])
    plsc.subcore_barrier()

    def body(j, carry):
        cidx = wid + NW * j

        @pl.when(cidx < NCHUNK)
        def _():
            base = cidx * CH
            pltpu.sync_copy(rowi.at[pl.ds(base, CH)], ib)
            pltpu.sync_copy(ob, acc.at[ib], add=True)

        return carry

    lax.fori_loop(0, NTRIP, body, 0)
    plsc.subcore_barrier()
    pltpu.sync_copy(acc.at[pl.ds(rbase, ROWS_PER_TILE)],
                    dg.at[cid, pl.ds(rbase, ROWS_PER_TILE)])


def _run_deg(rowi, ones8, zz8):
    f32 = jnp.float32
    fn = pl.kernel(
        _deg_body,
        out_type=[
            jax.ShapeDtypeStruct((NC, N, 8), f32),
        ],
        mesh=plsc.VectorSubcoreMesh(core_axis_name="c", subcore_axis_name="s", num_cores=NC, num_subcores=NS),
        compiler_params=pltpu.CompilerParams(needs_layout_passes=False, use_tc_tiling_on_sc=False),
        scratch_types=[
            pltpu.VMEM_SHARED((N, 8), f32),
            pltpu.VMEM((CH,), jnp.int32),
            pltpu.VMEM((CH, 8), f32),
        ],
    )
    return fn(rowi, ones8, zz8)


def _update_body(x_ref, hb_ref, ps_ref, dg_ref, mW4r, mb4r, An, Bn,
                 xn_ref, hbn_ref, nA_ref, nB_ref):
    f32 = jnp.float32
    ps = ps_ref[0] + ps_ref[1]
    pzs = ps[0]
    xn_ref[...] = x_ref[...] + ps[1][:, :3] * (1.0 / NORM)
    deg = dg_ref[0][:, 0:1] + dg_ref[1][:, 0:1]
    hbn = hb_ref[...] + (jnp.dot(pzs, mW4r[...], preferred_element_type=f32)
                         + deg * mb4r[...]) * (1.0 / NORM)
    hbn_ref[...] = hbn
    nA_ref[...] = jnp.dot(hbn, An[...], preferred_element_type=f32)
    nB_ref[...] = jnp.dot(hbn, Bn[...], preferred_element_type=f32)


def _run_update(x, hb, ps, dg, ws):
    f32 = jnp.float32
    nb = 10
    blk = N // nb
    full = lambda s: pl.BlockSpec(s, lambda i: (0,) * len(s))
    return pl.pallas_call(
        _update_body,
        grid=(nb,),
        in_specs=[
            pl.BlockSpec((blk, 3), lambda i: (i, 0)),
            pl.BlockSpec((blk, H), lambda i: (i, 0)),
            pl.BlockSpec((NC, 2, blk, M), lambda i: (0, 0, i, 0)),
            pl.BlockSpec((NC, blk, 8), lambda i: (0, i, 0)),
            full((M, H)), full((1, H)), full((H, M)), full((H, M)),
        ],
        out_specs=[
            pl.BlockSpec((blk, 3), lambda i: (i, 0)),
            pl.BlockSpec((blk, H), lambda i: (i, 0)),
            pl.BlockSpec((blk, M), lambda i: (i, 0)),
            pl.BlockSpec((blk, M), lambda i: (i, 0)),
        ],
        out_shape=[
            jax.ShapeDtypeStruct((N, 3), f32),
            jax.ShapeDtypeStruct((N, H), f32),
            jax.ShapeDtypeStruct((N, M), f32),
            jax.ShapeDtypeStruct((N, M), f32),
        ],
        interpret=_INTERPRET,
    )(x, hb, ps, dg, *ws)


def _final_body(x_ref, hb_ref, ps_ref, dg_ref, mW4r, mb4r, oW1, ob1, tcco_ref,
                xn_ref, ho_ref):
    f32 = jnp.float32
    ps = ps_ref[0] + ps_ref[1]
    pzs = ps[0]
    xn_ref[...] = x_ref[...] + ps[1][:, :3] * (1.0 / NORM)
    deg = dg_ref[0][:, 0:1] + dg_ref[1][:, 0:1]
    hbn = hb_ref[...] + (jnp.dot(pzs, mW4r[...], preferred_element_type=f32)
                         + deg * mb4r[...]) * (1.0 / NORM)
    ho_ref[...] = (jnp.dot(hbn, oW1[...], preferred_element_type=f32)
                   + ob1[...] - tcco_ref[...])


def _run_final(x, hb, ps, dg, ws, tcco):
    f32 = jnp.float32
    nb = 10
    blk = N // nb
    full = lambda s: pl.BlockSpec(s, lambda i: (0,) * len(s))
    return pl.pallas_call(
        _final_body,
        grid=(nb,),
        in_specs=[
            pl.BlockSpec((blk, 3), lambda i: (i, 0)),
            pl.BlockSpec((blk, H), lambda i: (i, 0)),
            pl.BlockSpec((NC, 2, blk, M), lambda i: (0, 0, i, 0)),
            pl.BlockSpec((NC, blk, 8), lambda i: (0, i, 0)),
            full((M, H)), full((1, H)), full((H, 5)), full((1, 5)),
            pl.BlockSpec((blk, 5), lambda i: (i, 0)),
        ],
        out_specs=[
            pl.BlockSpec((blk, 3), lambda i: (i, 0)),
            pl.BlockSpec((blk, 5), lambda i: (i, 0)),
        ],
        out_shape=[
            jax.ShapeDtypeStruct((N, 3), f32),
            jax.ShapeDtypeStruct((N, 5), f32),
        ],
        interpret=_INTERPRET,
    )(x, hb, ps, dg, *ws, tcco)


def kernel(x, h, c, batch, edge_index, t,
           hW1, hb1, hW2, hb2, dW1, db1, dW2, db2,
           tW1, tb1, tW2, tb2, cW1, cb1, cW2, cb2,
           sW1, sb1, sW2, sb2, oW1, ob1,
           mW1, mb1, mW2, mb2, mW3, mb3, mW4, mb4):
    f32 = jnp.float32
    r1 = lambda v: v.reshape(1, -1)
    row = edge_index[0]
    col = edge_index[1]
    batchf = batch.astype(f32).reshape(N, 1)

    A = [mW1[r][:H] for r in range(3)]
    B = [mW1[r][H:2 * H] for r in range(3)]
    Dd = [mW1[r][2 * H:] for r in range(3)]
    Dfold = [dW2 @ Dd[r] for r in range(3)]
    bfold = [r1(db2 @ Dd[r] + mb1[r]) for r in range(3)]
    S1f = [mW4[r] @ sW1 for r in range(3)]
    sb1f = [r1(mb4[r] @ sW1 + sb1) for r in range(3)]

    eye4 = jnp.eye(4, dtype=f32)
    bd = lambda W: jnp.kron(eye4, W)
    t4 = lambda b: jnp.tile(b, (1, 4))
    sel3 = jnp.zeros((128, 128), f32)
    bc4 = jnp.zeros((4, 128), f32)
    for j in range(4):
        sel3 = sel3.at[32 * j + 3, 32 * j:32 * j + 32].set(1.0)
        bc4 = bc4.at[j, 32 * j:32 * j + 32].set(1.0)

    hb, nA, nB, tcco = _run_prep(
        h, batchf, t, c,
        (hW1, r1(hb1), hW2, r1(hb2), tW1, r1(tb1), tW2, r1(tb2),
         cW1, r1(cb1), cW2, r1(cb2), oW1, r1(ob1), A[0], B[0]))

    zz = jnp.zeros((N, M), f32)
    sW2r = r1(sW2[:, 0])
    sb2r = sb2.reshape(1, 1)
    dW1r4 = t4(r1(dW1[0]))
    db1r4 = t4(r1(db1))

    dg, = _run_deg(row, jnp.ones((CH, 8), f32), jnp.zeros((N, 8), f32))

    for r in range(3):
        x0 = x[:, 0]
        x1 = x[:, 1]
        x2 = x[:, 2]
        gA, gB, dif = _run_gather(nA, nB, x0, x1, x2, row, col)
        S1fj = [jnp.zeros((128, H), f32).at[32 * j:32 * j + 32].set(S1f[r])
                for j in range(4)]
        zwp, = _run_edge(gA.reshape(EP, 128), gB.reshape(EP, 128),
                         dif.reshape(EP, 128),
                         (sel3, dW1r4, db1r4, bd(Dfold[r]), t4(bfold[r]),
                          bd(mW2[r]), t4(r1(mb2[r])), bd(mW3[r]), t4(r1(mb3[r])),
                          *S1fj, sb1f[r], sW2r, sb2r, bc4))
        ps, = _run_scatter(row, zwp.reshape(2, E, M), zz)
        if r < 2:
            x, hb, nA, nB = _run_update(
                x, hb, ps, dg, (mW4[r], r1(mb4[r]), A[r + 1], B[r + 1]))
        else:
            x, h_out = _run_final(
                x, hb, ps, dg, (mW4[r], r1(mb4[r]), oW1, r1(ob1)), tcco)
    return (x, h_out)


# speedup vs baseline: 14.1604x; 1.1412x over previous
import functools

import jax
import jax.numpy as jnp
from jax import lax
from jax.experimental import pallas as pl
from jax.experimental.pallas import tpu as pltpu
from jax.experimental.pallas import tpu_sc as plsc

N = 10000
E = 320000
G = 64
H = 128
M = 32
NORM = 100.0

NC = 2
NS = 16
NW = NC * NS
CH = 512
NCHUNK = E // CH
NTRIP = (NCHUNK + NW - 1) // NW
ROWS_PER_TILE = N // NS
EP = E * M // 128

_INTERPRET = False


def _silu(v):
    return v * jax.nn.sigmoid(v)


def _prep_body(h_ref, bf_ref, t_ref, c_ref,
               hW1, hb1, hW2, hb2, tW1, tb1, tW2, tb2, cW1, cb1, cW2, cb2,
               oW1, ob1, A0, B0,
               hb_ref, nA_ref, nB_ref, tcco_ref):
    f32 = jnp.float32
    tcc = (_silu(jnp.dot(t_ref[...], tW1[...], preferred_element_type=f32) + tb1[...])
           @ tW2[...] + tb2[...])
    tcc = tcc + (_silu(jnp.dot(c_ref[...], cW1[...], preferred_element_type=f32) + cb1[...])
                 @ cW2[...] + cb2[...])
    h0 = (_silu(jnp.dot(h_ref[...], hW1[...], preferred_element_type=f32) + hb1[...])
          @ hW2[...] + hb2[...])
    gids = lax.broadcasted_iota(jnp.int32, (h_ref.shape[0], G), 1).astype(f32)
    oh = (bf_ref[...] == gids).astype(f32)
    hb = h0 + jnp.dot(oh, tcc, preferred_element_type=f32)
    hb_ref[...] = hb
    tcco_ref[...] = jnp.dot(oh, jnp.dot(tcc, oW1[...], preferred_element_type=f32),
                            preferred_element_type=f32)
    nA_ref[...] = jnp.dot(hb, A0[...], preferred_element_type=f32)
    nB_ref[...] = jnp.dot(hb, B0[...], preferred_element_type=f32)


def _run_prep(h, batchf, t, c, ws):
    f32 = jnp.float32
    nb = 10
    blk = N // nb
    full = lambda s: pl.BlockSpec(s, lambda i: (0,) * len(s))
    return pl.pallas_call(
        _prep_body,
        grid=(nb,),
        in_specs=[
            pl.BlockSpec((blk, 5), lambda i: (i, 0)),
            pl.BlockSpec((blk, 1), lambda i: (i, 0)),
            full((G, 6)), full((G, 5)),
            full((5, H)), full((1, H)), full((H, H)), full((1, H)),
            full((6, H)), full((1, H)), full((H, H)), full((1, H)),
            full((5, H)), full((1, H)), full((H, H)), full((1, H)),
            full((H, 5)), full((1, 5)), full((H, M)), full((H, M)),
        ],
        out_specs=[
            pl.BlockSpec((blk, H), lambda i: (i, 0)),
            pl.BlockSpec((blk, M), lambda i: (i, 0)),
            pl.BlockSpec((blk, M), lambda i: (i, 0)),
            pl.BlockSpec((blk, 5), lambda i: (i, 0)),
        ],
        out_shape=[
            jax.ShapeDtypeStruct((N, H), f32),
            jax.ShapeDtypeStruct((N, M), f32),
            jax.ShapeDtypeStruct((N, M), f32),
            jax.ShapeDtypeStruct((N, 5), f32),
        ],
        interpret=_INTERPRET,
    )(h, batchf, t, c, *ws)


def _gather_body(nA, nB, x0, x1, x2, rowi, coli,
                 gA, gB, dif,
                 x0v, x1v, x2v, ibr, ibc, bufA, bufB, bufD, sem1, sem2):
    wid = lax.axis_index("s") * NC + lax.axis_index("c")
    pltpu.sync_copy(x0, x0v)
    pltpu.sync_copy(x1, x1v)
    pltpu.sync_copy(x2, x2v)
    iota = lax.iota(jnp.int32, 16)

    z16 = jnp.zeros((16,), jnp.float32)

    def zbody(j, carry):
        idx = j * 16 + iota
        plsc.store_scatter(bufD, [idx >> 5, idx & 31], z16)
        return carry

    lax.fori_loop(0, (CH * M) // 16, zbody, 0)

    def body(j, carry):
        cidx = wid + NW * j

        @pl.when(cidx < NCHUNK)
        def _():
            base = cidx * CH
            pltpu.sync_copy(rowi.at[pl.ds(base, CH)], ibr)
            pltpu.sync_copy(coli.at[pl.ds(base, CH)], ibc)
            cpA = pltpu.async_copy(nA.at[ibr], bufA, sem1)
            cpB = pltpu.async_copy(nB.at[ibc], bufB, sem2)
            for i in range(CH // 16):
                r16 = ibr[pl.ds(i * 16, 16)]
                c16 = ibc[pl.ds(i * 16, 16)]
                d0 = plsc.load_gather(x0v, [r16]) - plsc.load_gather(x0v, [c16])
                d1 = plsc.load_gather(x1v, [r16]) - plsc.load_gather(x1v, [c16])
                d2 = plsc.load_gather(x2v, [r16]) - plsc.load_gather(x2v, [c16])
                sq = d0 * d0 + d1 * d1 + d2 * d2
                er = iota + (i * 16)
                for comp, val in ((0, d0), (1, d1), (2, d2), (3, sq)):
                    cc = jnp.full((16,), comp, jnp.int32)
                    plsc.store_scatter(bufD, [er, cc], val)
            cpA.wait()
            cpB.wait()
            pltpu.sync_copy(bufA, gA.at[pl.ds(base, CH)])
            pltpu.sync_copy(bufB, gB.at[pl.ds(base, CH)])
            pltpu.sync_copy(bufD, dif.at[pl.ds(base, CH)])

        return carry

    lax.fori_loop(0, NTRIP, body, 0)


def _run_gather(nA, nB, x0, x1, x2, rowi, coli):
    f32 = jnp.float32
    fn = pl.kernel(
        _gather_body,
        out_type=[
            jax.ShapeDtypeStruct((E, M), f32),
            jax.ShapeDtypeStruct((E, M), f32),
            jax.ShapeDtypeStruct((E, M), f32),
        ],
        mesh=plsc.VectorSubcoreMesh(core_axis_name="c", subcore_axis_name="s", num_cores=NC, num_subcores=NS),
        compiler_params=pltpu.CompilerParams(needs_layout_passes=False, use_tc_tiling_on_sc=False),
        scratch_types=[
            pltpu.VMEM((N,), f32), pltpu.VMEM((N,), f32), pltpu.VMEM((N,), f32),
            pltpu.VMEM((CH,), jnp.int32), pltpu.VMEM((CH,), jnp.int32),
            pltpu.VMEM((CH, M), f32), pltpu.VMEM((CH, M), f32),
            pltpu.VMEM((CH, M), f32),
            pltpu.SemaphoreType.DMA, pltpu.SemaphoreType.DMA,
        ],
    )
    return fn(nA, nB, x0, x1, x2, rowi, coli)


def _edge_body(gA, gB, dfp, SEL3, dW1r4, db1r4, Dbd, bf4, W2bd, b24, W3bd, b34,
               S1f0, S1f1, S1f2, S1f3, sb1f, sW2r, sb2r, BC4, zw_ref):
    f32 = jnp.float32
    df = dfp[...]
    sqb = jnp.dot(df, SEL3[...], preferred_element_type=f32)
    dist = jnp.sqrt(jnp.maximum(sqb, 1e-12))
    d2 = _silu(dist * dW1r4[...] + db1r4[...])
    pre = (gA[...] + gB[...] + jnp.dot(d2, Dbd[...], preferred_element_type=f32)
           + bf4[...])
    m1 = _silu(pre)
    m2 = _silu(jnp.dot(m1, W2bd[...], preferred_element_type=f32) + b24[...])
    z = _silu(jnp.dot(m2, W3bd[...], preferred_element_type=f32) + b34[...])
    s_cols = []
    for Sj in (S1f0, S1f1, S1f2, S1f3):
        uj = _silu(jnp.dot(z, Sj[...], preferred_element_type=f32) + sb1f[...])
        s_cols.append(jnp.sum(uj * sW2r[...], axis=1, keepdims=True))
    w4 = jnp.tanh(jnp.concatenate(s_cols, axis=1) + sb2r[...])
    wb = jnp.dot(w4, BC4[...], preferred_element_type=f32)
    zw_ref[0] = z
    zw_ref[1] = df * wb


def _run_edge(gA, gB, dfp, ws):
    f32 = jnp.float32
    nb = 80
    blk = EP // nb
    full = lambda s: pl.BlockSpec(s, lambda i: (0,) * len(s))
    return pl.pallas_call(
        _edge_body,
        grid=(nb,),
        in_specs=[
            pl.BlockSpec((blk, 128), lambda i: (i, 0)),
            pl.BlockSpec((blk, 128), lambda i: (i, 0)),
            pl.BlockSpec((blk, 128), lambda i: (i, 0)),
            full((128, 128)), full((1, 128)), full((1, 128)),
            full((128, 128)), full((1, 128)),
            full((128, 128)), full((1, 128)), full((128, 128)), full((1, 128)),
            full((128, H)), full((128, H)), full((128, H)), full((128, H)),
            full((1, H)), full((1, H)), full((1, 1)),
            full((4, 128)),
        ],
        out_specs=[
            pl.BlockSpec((2, blk, 128), lambda i: (0, i, 0)),
        ],
        out_shape=[
            jax.ShapeDtypeStruct((2, EP, 128), f32),
        ],
        interpret=_INTERPRET,
    )(gA, gB, dfp, *ws)


def _scatter_body(rowi, zw, zz,
                  ps,
                  accZ, accW, ib, zbuf, wbuf):
    cid = lax.axis_index("c")
    sid = lax.axis_index("s")
    wid = sid * NC + cid
    rbase = sid * ROWS_PER_TILE
    pltpu.sync_copy(zz.at[pl.ds(rbase, ROWS_PER_TILE)],
                    accZ.at[pl.ds(rbase, ROWS_PER_TILE)])
    pltpu.sync_copy(zz.at[pl.ds(rbase, ROWS_PER_TILE)],
                    accW.at[pl.ds(rbase, ROWS_PER_TILE)])
    plsc.subcore_barrier()

    def body(j, carry):
        cidx = wid + NW * j

        @pl.when(cidx < NCHUNK)
        def _():
            base = cidx * CH
            pltpu.sync_copy(rowi.at[pl.ds(base, CH)], ib)
            pltpu.sync_copy(zw.at[0, pl.ds(base, CH)], zbuf)
            pltpu.sync_copy(zw.at[1, pl.ds(base, CH)], wbuf)
            pltpu.sync_copy(zbuf, accZ.at[ib], add=True)
            pltpu.sync_copy(wbuf, accW.at[ib], add=True)

        return carry

    lax.fori_loop(0, NTRIP, body, 0)
    plsc.subcore_barrier()
    pltpu.sync_copy(accZ.at[pl.ds(rbase, ROWS_PER_TILE)],
                    ps.at[cid, 0, pl.ds(rbase, ROWS_PER_TILE)])
    pltpu.sync_copy(accW.at[pl.ds(rbase, ROWS_PER_TILE)],
                    ps.at[cid, 1, pl.ds(rbase, ROWS_PER_TILE)])


def _run_scatter(rowi, zw, zz):
    f32 = jnp.float32
    fn = pl.kernel(
        _scatter_body,
        out_type=[
            jax.ShapeDtypeStruct((NC, 2, N, M), f32),
        ],
        mesh=plsc.VectorSubcoreMesh(core_axis_name="c", subcore_axis_name="s", num_cores=NC, num_subcores=NS),
        compiler_params=pltpu.CompilerParams(needs_layout_passes=False, use_tc_tiling_on_sc=False),
        scratch_types=[
            pltpu.VMEM_SHARED((N, M), f32), pltpu.VMEM_SHARED((N, M), f32),
            pltpu.VMEM((CH,), jnp.int32),
            pltpu.VMEM((CH, M), f32), pltpu.VMEM((CH, M), f32),
        ],
    )
    return fn(rowi, zw, zz)


def _deg_body(rowi, ones8, zz8,
              dg,
              acc, ib, ob):
    cid = lax.axis_index("c")
    sid = lax.axis_index("s")
    wid = sid * NC + cid
    rbase = sid * ROWS_PER_TILE
    pltpu.sync_copy(ones8, ob)
    pltpu.sync_copy(zz8.at[pl.ds(rbase, ROWS_PER_TILE)],
                    acc.at[pl.ds(rbase, ROWS_PER_TILE)])
    plsc.subcore_barrier()

    def body(j, carry):
        cidx = wid + NW * j

        @pl.when(cidx < NCHUNK)
        def _():
            base = cidx * CH
            pltpu.sync_copy(rowi.at[pl.ds(base, CH)], ib)
            pltpu.sync_copy(ob, acc.at[ib], add=True)

        return carry

    lax.fori_loop(0, NTRIP, body, 0)
    plsc.subcore_barrier()
    pltpu.sync_copy(acc.at[pl.ds(rbase, ROWS_PER_TILE)],
                    dg.at[cid, pl.ds(rbase, ROWS_PER_TILE)])


def _run_deg(rowi, ones8, zz8):
    f32 = jnp.float32
    fn = pl.kernel(
        _deg_body,
        out_type=[
            jax.ShapeDtypeStruct((NC, N, 8), f32),
        ],
        mesh=plsc.VectorSubcoreMesh(core_axis_name="c", subcore_axis_name="s", num_cores=NC, num_subcores=NS),
        compiler_params=pltpu.CompilerParams(needs_layout_passes=False, use_tc_tiling_on_sc=False),
        scratch_types=[
            pltpu.VMEM_SHARED((N, 8), f32),
            pltpu.VMEM((CH,), jnp.int32),
            pltpu.VMEM((CH, 8), f32),
        ],
    )
    return fn(rowi, ones8, zz8)


def _update_body(x_ref, hb_ref, ps_ref, dg_ref, mW4r, mb4r, An, Bn,
                 xn_ref, hbn_ref, nA_ref, nB_ref):
    f32 = jnp.float32
    ps = ps_ref[0] + ps_ref[1]
    pzs = ps[0]
    xn_ref[...] = x_ref[...] + ps[1][:, :3] * (1.0 / NORM)
    deg = dg_ref[0][:, 0:1] + dg_ref[1][:, 0:1]
    hbn = hb_ref[...] + (jnp.dot(pzs, mW4r[...], preferred_element_type=f32)
                         + deg * mb4r[...]) * (1.0 / NORM)
    hbn_ref[...] = hbn
    nA_ref[...] = jnp.dot(hbn, An[...], preferred_element_type=f32)
    nB_ref[...] = jnp.dot(hbn, Bn[...], preferred_element_type=f32)


def _run_update(x, hb, ps, dg, ws):
    f32 = jnp.float32
    nb = 10
    blk = N // nb
    full = lambda s: pl.BlockSpec(s, lambda i: (0,) * len(s))
    return pl.pallas_call(
        _update_body,
        grid=(nb,),
        in_specs=[
            pl.BlockSpec((blk, 3), lambda i: (i, 0)),
            pl.BlockSpec((blk, H), lambda i: (i, 0)),
            pl.BlockSpec((NC, 2, blk, M), lambda i: (0, 0, i, 0)),
            pl.BlockSpec((NC, blk, 8), lambda i: (0, i, 0)),
            full((M, H)), full((1, H)), full((H, M)), full((H, M)),
        ],
        out_specs=[
            pl.BlockSpec((blk, 3), lambda i: (i, 0)),
            pl.BlockSpec((blk, H), lambda i: (i, 0)),
            pl.BlockSpec((blk, M), lambda i: (i, 0)),
            pl.BlockSpec((blk, M), lambda i: (i, 0)),
        ],
        out_shape=[
            jax.ShapeDtypeStruct((N, 3), f32),
            jax.ShapeDtypeStruct((N, H), f32),
            jax.ShapeDtypeStruct((N, M), f32),
            jax.ShapeDtypeStruct((N, M), f32),
        ],
        interpret=_INTERPRET,
    )(x, hb, ps, dg, *ws)


def _final_body(x_ref, hb_ref, ps_ref, dg_ref, mW4r, mb4r, oW1, ob1, tcco_ref,
                xn_ref, ho_ref):
    f32 = jnp.float32
    ps = ps_ref[0] + ps_ref[1]
    pzs = ps[0]
    xn_ref[...] = x_ref[...] + ps[1][:, :3] * (1.0 / NORM)
    deg = dg_ref[0][:, 0:1] + dg_ref[1][:, 0:1]
    hbn = hb_ref[...] + (jnp.dot(pzs, mW4r[...], preferred_element_type=f32)
                         + deg * mb4r[...]) * (1.0 / NORM)
    ho_ref[...] = (jnp.dot(hbn, oW1[...], preferred_element_type=f32)
                   + ob1[...] - tcco_ref[...])


def _run_final(x, hb, ps, dg, ws, tcco):
    f32 = jnp.float32
    nb = 10
    blk = N // nb
    full = lambda s: pl.BlockSpec(s, lambda i: (0,) * len(s))
    return pl.pallas_call(
        _final_body,
        grid=(nb,),
        in_specs=[
            pl.BlockSpec((blk, 3), lambda i: (i, 0)),
            pl.BlockSpec((blk, H), lambda i: (i, 0)),
            pl.BlockSpec((NC, 2, blk, M), lambda i: (0, 0, i, 0)),
            pl.BlockSpec((NC, blk, 8), lambda i: (0, i, 0)),
            full((M, H)), full((1, H)), full((H, 5)), full((1, 5)),
            pl.BlockSpec((blk, 5), lambda i: (i, 0)),
        ],
        out_specs=[
            pl.BlockSpec((blk, 3), lambda i: (i, 0)),
            pl.BlockSpec((blk, 5), lambda i: (i, 0)),
        ],
        out_shape=[
            jax.ShapeDtypeStruct((N, 3), f32),
            jax.ShapeDtypeStruct((N, 5), f32),
        ],
        interpret=_INTERPRET,
    )(x, hb, ps, dg, *ws, tcco)


def kernel(x, h, c, batch, edge_index, t,
           hW1, hb1, hW2, hb2, dW1, db1, dW2, db2,
           tW1, tb1, tW2, tb2, cW1, cb1, cW2, cb2,
           sW1, sb1, sW2, sb2, oW1, ob1,
           mW1, mb1, mW2, mb2, mW3, mb3, mW4, mb4):
    f32 = jnp.float32
    r1 = lambda v: v.reshape(1, -1)
    row = edge_index[0]
    col = edge_index[1]
    batchf = batch.astype(f32).reshape(N, 1)

    A = [mW1[r][:H] for r in range(3)]
    B = [mW1[r][H:2 * H] for r in range(3)]
    Dd = [mW1[r][2 * H:] for r in range(3)]
    Dfold = [dW2 @ Dd[r] for r in range(3)]
    bfold = [r1(db2 @ Dd[r] + mb1[r]) for r in range(3)]
    S1f = [mW4[r] @ sW1 for r in range(3)]
    sb1f = [r1(mb4[r] @ sW1 + sb1) for r in range(3)]

    eye4 = jnp.eye(4, dtype=f32)
    bd = lambda W: jnp.kron(eye4, W)
    t4 = lambda b: jnp.tile(b, (1, 4))
    sel3 = jnp.zeros((128, 128), f32)
    bc4 = jnp.zeros((4, 128), f32)
    for j in range(4):
        sel3 = sel3.at[32 * j + 3, 32 * j:32 * j + 32].set(1.0)
        bc4 = bc4.at[j, 32 * j:32 * j + 32].set(1.0)

    hb, nA, nB, tcco = _run_prep(
        h, batchf, t, c,
        (hW1, r1(hb1), hW2, r1(hb2), tW1, r1(tb1), tW2, r1(tb2),
         cW1, r1(cb1), cW2, r1(cb2), oW1, r1(ob1), A[0], B[0]))

    zz = jnp.zeros((N, M), f32)
    sW2r = r1(sW2[:, 0])
    sb2r = sb2.reshape(1, 1)
    dW1r4 = t4(r1(dW1[0]))
    db1r4 = t4(r1(db1))

    dg, = _run_deg(row, jnp.ones((CH, 8), f32), jnp.zeros((N, 8), f32))

    for r in range(3):
        x0 = x[:, 0]
        x1 = x[:, 1]
        x2 = x[:, 2]
        gA, gB, dif = _run_gather(nA, nB, x0, x1, x2, row, col)
        S1fj = [jnp.zeros((128, H), f32).at[32 * j:32 * j + 32].set(S1f[r])
                for j in range(4)]
        zwp, = _run_edge(gA.reshape(EP, 128), gB.reshape(EP, 128),
                         dif.reshape(EP, 128),
                         (sel3, dW1r4, db1r4, bd(Dfold[r]), t4(bfold[r]),
                          bd(mW2[r]), t4(r1(mb2[r])), bd(mW3[r]), t4(r1(mb3[r])),
                          *S1fj, sb1f[r], sW2r, sb2r, bc4))
        ps, = _run_scatter(row, zwp.reshape(2, E, M), zz)
        if r < 2:
            x, hb, nA, nB = _run_update(
                x, hb, ps, dg, (mW4[r], r1(mb4[r]), A[r + 1], B[r + 1]))
        else:
            x, h_out = _run_final(
                x, hb, ps, dg, (mW4[r], r1(mb4[r]), oW1, r1(ob1)), tcco)
    return (x, h_out)


# speedup vs baseline: 14.9962x; 1.0590x over previous
import functools

import jax
import jax.numpy as jnp
from jax import lax
from jax.experimental import pallas as pl
from jax.experimental.pallas import tpu as pltpu
from jax.experimental.pallas import tpu_sc as plsc

N = 10000
E = 320000
G = 64
H = 128
M = 32
NORM = 100.0

NC = 2
NS = 16
NW = NC * NS
CH = 1024
NCHUNK = E // CH
NTRIP = (NCHUNK + NW - 1) // NW
ROWS_PER_TILE = N // NS
EP = E * M // 128

_INTERPRET = False


def _silu(v):
    return v * jax.nn.sigmoid(v)


def _prep_body(h_ref, bf_ref, t_ref, c_ref,
               hW1, hb1, hW2, hb2, tW1, tb1, tW2, tb2, cW1, cb1, cW2, cb2,
               oW1, ob1, A0, B0,
               hb_ref, nA_ref, nB_ref, tcco_ref):
    f32 = jnp.float32
    tcc = (_silu(jnp.dot(t_ref[...], tW1[...], preferred_element_type=f32) + tb1[...])
           @ tW2[...] + tb2[...])
    tcc = tcc + (_silu(jnp.dot(c_ref[...], cW1[...], preferred_element_type=f32) + cb1[...])
                 @ cW2[...] + cb2[...])
    h0 = (_silu(jnp.dot(h_ref[...], hW1[...], preferred_element_type=f32) + hb1[...])
          @ hW2[...] + hb2[...])
    gids = lax.broadcasted_iota(jnp.int32, (h_ref.shape[0], G), 1).astype(f32)
    oh = (bf_ref[...] == gids).astype(f32)
    hb = h0 + jnp.dot(oh, tcc, preferred_element_type=f32)
    hb_ref[...] = hb
    tcco_ref[...] = jnp.dot(oh, jnp.dot(tcc, oW1[...], preferred_element_type=f32),
                            preferred_element_type=f32)
    nA_ref[...] = jnp.dot(hb, A0[...], preferred_element_type=f32)
    nB_ref[...] = jnp.dot(hb, B0[...], preferred_element_type=f32)


def _run_prep(h, batchf, t, c, ws):
    f32 = jnp.float32
    nb = 10
    blk = N // nb
    full = lambda s: pl.BlockSpec(s, lambda i: (0,) * len(s))
    return pl.pallas_call(
        _prep_body,
        grid=(nb,),
        in_specs=[
            pl.BlockSpec((blk, 5), lambda i: (i, 0)),
            pl.BlockSpec((blk, 1), lambda i: (i, 0)),
            full((G, 6)), full((G, 5)),
            full((5, H)), full((1, H)), full((H, H)), full((1, H)),
            full((6, H)), full((1, H)), full((H, H)), full((1, H)),
            full((5, H)), full((1, H)), full((H, H)), full((1, H)),
            full((H, 5)), full((1, 5)), full((H, M)), full((H, M)),
        ],
        out_specs=[
            pl.BlockSpec((blk, H), lambda i: (i, 0)),
            pl.BlockSpec((blk, M), lambda i: (i, 0)),
            pl.BlockSpec((blk, M), lambda i: (i, 0)),
            pl.BlockSpec((blk, 5), lambda i: (i, 0)),
        ],
        out_shape=[
            jax.ShapeDtypeStruct((N, H), f32),
            jax.ShapeDtypeStruct((N, M), f32),
            jax.ShapeDtypeStruct((N, M), f32),
            jax.ShapeDtypeStruct((N, 5), f32),
        ],
        interpret=_INTERPRET,
    )(h, batchf, t, c, *ws)


def _gather_body(nA, nB, x0, x1, x2, rowi, coli,
                 gA, gB, dif,
                 x0v, x1v, x2v, ibr, ibc, bufA, bufB, bufD, sem1, sem2):
    wid = lax.axis_index("s") * NC + lax.axis_index("c")
    pltpu.sync_copy(x0, x0v)
    pltpu.sync_copy(x1, x1v)
    pltpu.sync_copy(x2, x2v)
    iota = lax.iota(jnp.int32, 16)

    z16 = jnp.zeros((16,), jnp.float32)

    def zbody(j, carry):
        idx = j * 16 + iota
        plsc.store_scatter(bufD, [idx >> 5, idx & 31], z16)
        return carry

    lax.fori_loop(0, (CH * M) // 16, zbody, 0)

    def body(j, carry):
        cidx = wid + NW * j

        @pl.when(cidx < NCHUNK)
        def _():
            base = cidx * CH
            pltpu.sync_copy(rowi.at[pl.ds(base, CH)], ibr)
            pltpu.sync_copy(coli.at[pl.ds(base, CH)], ibc)
            cpA = pltpu.async_copy(nA.at[ibr], bufA, sem1)
            cpB = pltpu.async_copy(nB.at[ibc], bufB, sem2)
            for i in range(CH // 16):
                r16 = ibr[pl.ds(i * 16, 16)]
                c16 = ibc[pl.ds(i * 16, 16)]
                d0 = plsc.load_gather(x0v, [r16]) - plsc.load_gather(x0v, [c16])
                d1 = plsc.load_gather(x1v, [r16]) - plsc.load_gather(x1v, [c16])
                d2 = plsc.load_gather(x2v, [r16]) - plsc.load_gather(x2v, [c16])
                sq = d0 * d0 + d1 * d1 + d2 * d2
                er = iota + (i * 16)
                for comp, val in ((0, d0), (1, d1), (2, d2), (3, sq)):
                    cc = jnp.full((16,), comp, jnp.int32)
                    plsc.store_scatter(bufD, [er, cc], val)
            cpA.wait()
            cpB.wait()
            pltpu.sync_copy(bufA, gA.at[pl.ds(base, CH)])
            pltpu.sync_copy(bufB, gB.at[pl.ds(base, CH)])
            pltpu.sync_copy(bufD, dif.at[pl.ds(base, CH)])

        return carry

    lax.fori_loop(0, NTRIP, body, 0)


def _run_gather(nA, nB, x0, x1, x2, rowi, coli):
    f32 = jnp.float32
    fn = pl.kernel(
        _gather_body,
        out_type=[
            jax.ShapeDtypeStruct((E, M), f32),
            jax.ShapeDtypeStruct((E, M), f32),
            jax.ShapeDtypeStruct((E, M), f32),
        ],
        mesh=plsc.VectorSubcoreMesh(core_axis_name="c", subcore_axis_name="s", num_cores=NC, num_subcores=NS),
        compiler_params=pltpu.CompilerParams(needs_layout_passes=False, use_tc_tiling_on_sc=False),
        scratch_types=[
            pltpu.VMEM((N,), f32), pltpu.VMEM((N,), f32), pltpu.VMEM((N,), f32),
            pltpu.VMEM((CH,), jnp.int32), pltpu.VMEM((CH,), jnp.int32),
            pltpu.VMEM((CH, M), f32), pltpu.VMEM((CH, M), f32),
            pltpu.VMEM((CH, M), f32),
            pltpu.SemaphoreType.DMA, pltpu.SemaphoreType.DMA,
        ],
    )
    return fn(nA, nB, x0, x1, x2, rowi, coli)


def _edge_body(gA, gB, dfp, SEL3, dW1r4, db1r4, Dbd, bf4, W2bd, b24, W3bd, b34,
               S1f0, S1f1, S1f2, S1f3, sb1f, sW2r, sb2r, BC4, zw_ref):
    f32 = jnp.float32
    df = dfp[...]
    sqb = jnp.dot(df, SEL3[...], preferred_element_type=f32)
    dist = jnp.sqrt(jnp.maximum(sqb, 1e-12))
    d2 = _silu(dist * dW1r4[...] + db1r4[...])
    pre = (gA[...] + gB[...] + jnp.dot(d2, Dbd[...], preferred_element_type=f32)
           + bf4[...])
    m1 = _silu(pre)
    m2 = _silu(jnp.dot(m1, W2bd[...], preferred_element_type=f32) + b24[...])
    z = _silu(jnp.dot(m2, W3bd[...], preferred_element_type=f32) + b34[...])
    s_cols = []
    for Sj in (S1f0, S1f1, S1f2, S1f3):
        uj = _silu(jnp.dot(z, Sj[...], preferred_element_type=f32) + sb1f[...])
        s_cols.append(jnp.sum(uj * sW2r[...], axis=1, keepdims=True))
    w4 = jnp.tanh(jnp.concatenate(s_cols, axis=1) + sb2r[...])
    wb = jnp.dot(w4, BC4[...], preferred_element_type=f32)
    zw_ref[0] = z
    zw_ref[1] = df * wb


def _run_edge(gA, gB, dfp, ws):
    f32 = jnp.float32
    nb = 80
    blk = EP // nb
    full = lambda s: pl.BlockSpec(s, lambda i: (0,) * len(s))
    return pl.pallas_call(
        _edge_body,
        grid=(nb,),
        in_specs=[
            pl.BlockSpec((blk, 128), lambda i: (i, 0)),
            pl.BlockSpec((blk, 128), lambda i: (i, 0)),
            pl.BlockSpec((blk, 128), lambda i: (i, 0)),
            full((128, 128)), full((1, 128)), full((1, 128)),
            full((128, 128)), full((1, 128)),
            full((128, 128)), full((1, 128)), full((128, 128)), full((1, 128)),
            full((128, H)), full((128, H)), full((128, H)), full((128, H)),
            full((1, H)), full((1, H)), full((1, 1)),
            full((4, 128)),
        ],
        out_specs=[
            pl.BlockSpec((2, blk, 128), lambda i: (0, i, 0)),
        ],
        out_shape=[
            jax.ShapeDtypeStruct((2, EP, 128), f32),
        ],
        interpret=_INTERPRET,
    )(gA, gB, dfp, *ws)


def _scatter_body(rowi, zw, zz,
                  ps,
                  accZ, accW, ib, zbuf, wbuf):
    cid = lax.axis_index("c")
    sid = lax.axis_index("s")
    wid = sid * NC + cid
    rbase = sid * ROWS_PER_TILE
    pltpu.sync_copy(zz.at[pl.ds(rbase, ROWS_PER_TILE)],
                    accZ.at[pl.ds(rbase, ROWS_PER_TILE)])
    pltpu.sync_copy(zz.at[pl.ds(rbase, ROWS_PER_TILE)],
                    accW.at[pl.ds(rbase, ROWS_PER_TILE)])
    plsc.subcore_barrier()

    def body(j, carry):
        cidx = wid + NW * j

        @pl.when(cidx < NCHUNK)
        def _():
            base = cidx * CH
            pltpu.sync_copy(rowi.at[pl.ds(base, CH)], ib)
            pltpu.sync_copy(zw.at[0, pl.ds(base, CH)], zbuf)
            pltpu.sync_copy(zw.at[1, pl.ds(base, CH)], wbuf)
            pltpu.sync_copy(zbuf, accZ.at[ib], add=True)
            pltpu.sync_copy(wbuf, accW.at[ib], add=True)

        return carry

    lax.fori_loop(0, NTRIP, body, 0)
    plsc.subcore_barrier()
    pltpu.sync_copy(accZ.at[pl.ds(rbase, ROWS_PER_TILE)],
                    ps.at[cid, 0, pl.ds(rbase, ROWS_PER_TILE)])
    pltpu.sync_copy(accW.at[pl.ds(rbase, ROWS_PER_TILE)],
                    ps.at[cid, 1, pl.ds(rbase, ROWS_PER_TILE)])


def _run_scatter(rowi, zw, zz):
    f32 = jnp.float32
    fn = pl.kernel(
        _scatter_body,
        out_type=[
            jax.ShapeDtypeStruct((NC, 2, N, M), f32),
        ],
        mesh=plsc.VectorSubcoreMesh(core_axis_name="c", subcore_axis_name="s", num_cores=NC, num_subcores=NS),
        compiler_params=pltpu.CompilerParams(needs_layout_passes=False, use_tc_tiling_on_sc=False),
        scratch_types=[
            pltpu.VMEM_SHARED((N, M), f32), pltpu.VMEM_SHARED((N, M), f32),
            pltpu.VMEM((CH,), jnp.int32),
            pltpu.VMEM((CH, M), f32), pltpu.VMEM((CH, M), f32),
        ],
    )
    return fn(rowi, zw, zz)


def _deg_body(rowi, ones8, zz8,
              dg,
              acc, ib, ob):
    cid = lax.axis_index("c")
    sid = lax.axis_index("s")
    wid = sid * NC + cid
    rbase = sid * ROWS_PER_TILE
    pltpu.sync_copy(ones8, ob)
    pltpu.sync_copy(zz8.at[pl.ds(rbase, ROWS_PER_TILE)],
                    acc.at[pl.ds(rbase, ROWS_PER_TILE)])
    plsc.subcore_barrier()

    def body(j, carry):
        cidx = wid + NW * j

        @pl.when(cidx < NCHUNK)
        def _():
            base = cidx * CH
            pltpu.sync_copy(rowi.at[pl.ds(base, CH)], ib)
            pltpu.sync_copy(ob, acc.at[ib], add=True)

        return carry

    lax.fori_loop(0, NTRIP, body, 0)
    plsc.subcore_barrier()
    pltpu.sync_copy(acc.at[pl.ds(rbase, ROWS_PER_TILE)],
                    dg.at[cid, pl.ds(rbase, ROWS_PER_TILE)])


def _run_deg(rowi, ones8, zz8):
    f32 = jnp.float32
    fn = pl.kernel(
        _deg_body,
        out_type=[
            jax.ShapeDtypeStruct((NC, N, 8), f32),
        ],
        mesh=plsc.VectorSubcoreMesh(core_axis_name="c", subcore_axis_name="s", num_cores=NC, num_subcores=NS),
        compiler_params=pltpu.CompilerParams(needs_layout_passes=False, use_tc_tiling_on_sc=False),
        scratch_types=[
            pltpu.VMEM_SHARED((N, 8), f32),
            pltpu.VMEM((CH,), jnp.int32),
            pltpu.VMEM((CH, 8), f32),
        ],
    )
    return fn(rowi, ones8, zz8)


def _update_body(x_ref, hb_ref, ps_ref, dg_ref, mW4r, mb4r, An, Bn,
                 xn_ref, hbn_ref, nA_ref, nB_ref):
    f32 = jnp.float32
    ps = ps_ref[0] + ps_ref[1]
    pzs = ps[0]
    xn_ref[...] = x_ref[...] + ps[1][:, :3] * (1.0 / NORM)
    deg = dg_ref[0][:, 0:1] + dg_ref[1][:, 0:1]
    hbn = hb_ref[...] + (jnp.dot(pzs, mW4r[...], preferred_element_type=f32)
                         + deg * mb4r[...]) * (1.0 / NORM)
    hbn_ref[...] = hbn
    nA_ref[...] = jnp.dot(hbn, An[...], preferred_element_type=f32)
    nB_ref[...] = jnp.dot(hbn, Bn[...], preferred_element_type=f32)


def _run_update(x, hb, ps, dg, ws):
    f32 = jnp.float32
    nb = 10
    blk = N // nb
    full = lambda s: pl.BlockSpec(s, lambda i: (0,) * len(s))
    return pl.pallas_call(
        _update_body,
        grid=(nb,),
        in_specs=[
            pl.BlockSpec((blk, 3), lambda i: (i, 0)),
            pl.BlockSpec((blk, H), lambda i: (i, 0)),
            pl.BlockSpec((NC, 2, blk, M), lambda i: (0, 0, i, 0)),
            pl.BlockSpec((NC, blk, 8), lambda i: (0, i, 0)),
            full((M, H)), full((1, H)), full((H, M)), full((H, M)),
        ],
        out_specs=[
            pl.BlockSpec((blk, 3), lambda i: (i, 0)),
            pl.BlockSpec((blk, H), lambda i: (i, 0)),
            pl.BlockSpec((blk, M), lambda i: (i, 0)),
            pl.BlockSpec((blk, M), lambda i: (i, 0)),
        ],
        out_shape=[
            jax.ShapeDtypeStruct((N, 3), f32),
            jax.ShapeDtypeStruct((N, H), f32),
            jax.ShapeDtypeStruct((N, M), f32),
            jax.ShapeDtypeStruct((N, M), f32),
        ],
        interpret=_INTERPRET,
    )(x, hb, ps, dg, *ws)


def _final_body(x_ref, hb_ref, ps_ref, dg_ref, mW4r, mb4r, oW1, ob1, tcco_ref,
                xn_ref, ho_ref):
    f32 = jnp.float32
    ps = ps_ref[0] + ps_ref[1]
    pzs = ps[0]
    xn_ref[...] = x_ref[...] + ps[1][:, :3] * (1.0 / NORM)
    deg = dg_ref[0][:, 0:1] + dg_ref[1][:, 0:1]
    hbn = hb_ref[...] + (jnp.dot(pzs, mW4r[...], preferred_element_type=f32)
                         + deg * mb4r[...]) * (1.0 / NORM)
    ho_ref[...] = (jnp.dot(hbn, oW1[...], preferred_element_type=f32)
                   + ob1[...] - tcco_ref[...])


def _run_final(x, hb, ps, dg, ws, tcco):
    f32 = jnp.float32
    nb = 10
    blk = N // nb
    full = lambda s: pl.BlockSpec(s, lambda i: (0,) * len(s))
    return pl.pallas_call(
        _final_body,
        grid=(nb,),
        in_specs=[
            pl.BlockSpec((blk, 3), lambda i: (i, 0)),
            pl.BlockSpec((blk, H), lambda i: (i, 0)),
            pl.BlockSpec((NC, 2, blk, M), lambda i: (0, 0, i, 0)),
            pl.BlockSpec((NC, blk, 8), lambda i: (0, i, 0)),
            full((M, H)), full((1, H)), full((H, 5)), full((1, 5)),
            pl.BlockSpec((blk, 5), lambda i: (i, 0)),
        ],
        out_specs=[
            pl.BlockSpec((blk, 3), lambda i: (i, 0)),
            pl.BlockSpec((blk, 5), lambda i: (i, 0)),
        ],
        out_shape=[
            jax.ShapeDtypeStruct((N, 3), f32),
            jax.ShapeDtypeStruct((N, 5), f32),
        ],
        interpret=_INTERPRET,
    )(x, hb, ps, dg, *ws, tcco)


def kernel(x, h, c, batch, edge_index, t,
           hW1, hb1, hW2, hb2, dW1, db1, dW2, db2,
           tW1, tb1, tW2, tb2, cW1, cb1, cW2, cb2,
           sW1, sb1, sW2, sb2, oW1, ob1,
           mW1, mb1, mW2, mb2, mW3, mb3, mW4, mb4):
    f32 = jnp.float32
    r1 = lambda v: v.reshape(1, -1)
    row = edge_index[0]
    col = edge_index[1]
    batchf = batch.astype(f32).reshape(N, 1)

    A = [mW1[r][:H] for r in range(3)]
    B = [mW1[r][H:2 * H] for r in range(3)]
    Dd = [mW1[r][2 * H:] for r in range(3)]
    Dfold = [dW2 @ Dd[r] for r in range(3)]
    bfold = [r1(db2 @ Dd[r] + mb1[r]) for r in range(3)]
    S1f = [mW4[r] @ sW1 for r in range(3)]
    sb1f = [r1(mb4[r] @ sW1 + sb1) for r in range(3)]

    eye4 = jnp.eye(4, dtype=f32)
    bd = lambda W: jnp.kron(eye4, W)
    t4 = lambda b: jnp.tile(b, (1, 4))
    sel3 = jnp.zeros((128, 128), f32)
    bc4 = jnp.zeros((4, 128), f32)
    for j in range(4):
        sel3 = sel3.at[32 * j + 3, 32 * j:32 * j + 32].set(1.0)
        bc4 = bc4.at[j, 32 * j:32 * j + 32].set(1.0)

    hb, nA, nB, tcco = _run_prep(
        h, batchf, t, c,
        (hW1, r1(hb1), hW2, r1(hb2), tW1, r1(tb1), tW2, r1(tb2),
         cW1, r1(cb1), cW2, r1(cb2), oW1, r1(ob1), A[0], B[0]))

    zz = jnp.zeros((N, M), f32)
    sW2r = r1(sW2[:, 0])
    sb2r = sb2.reshape(1, 1)
    dW1r4 = t4(r1(dW1[0]))
    db1r4 = t4(r1(db1))

    dg, = _run_deg(row, jnp.ones((CH, 8), f32), jnp.zeros((N, 8), f32))

    for r in range(3):
        x0 = x[:, 0]
        x1 = x[:, 1]
        x2 = x[:, 2]
        gA, gB, dif = _run_gather(nA, nB, x0, x1, x2, row, col)
        S1fj = [jnp.zeros((128, H), f32).at[32 * j:32 * j + 32].set(S1f[r])
                for j in range(4)]
        zwp, = _run_edge(gA.reshape(EP, 128), gB.reshape(EP, 128),
                         dif.reshape(EP, 128),
                         (sel3, dW1r4, db1r4, bd(Dfold[r]), t4(bfold[r]),
                          bd(mW2[r]), t4(r1(mb2[r])), bd(mW3[r]), t4(r1(mb3[r])),
                          *S1fj, sb1f[r], sW2r, sb2r, bc4))
        ps, = _run_scatter(row, zwp.reshape(2, E, M), zz)
        if r < 2:
            x, hb, nA, nB = _run_update(
                x, hb, ps, dg, (mW4[r], r1(mb4[r]), A[r + 1], B[r + 1]))
        else:
            x, h_out = _run_final(
                x, hb, ps, dg, (mW4[r], r1(mb4[r]), oW1, r1(ob1)), tcco)
    return (x, h_out)
